# baseline (speedup 1.0000x reference)
import jax
import jax.numpy as jnp
from jax import lax
from jax.experimental import pallas as pl
from jax.experimental.pallas import tpu as pltpu
from jax.experimental.pallas import tpu_sc as plsc

N = 10000
E = 320000
IN_DIM = 128
HID = 64
HEADS = 8
EPS = 1e-5

NP = 10240
ET = E + N
CH = 128
EP = 331776
SENT = NP - 1
NEG = -1e30

NC = 2
NS = 16
HALF = NP // 2
OB = HALF + 128
STRIPE = NP // NS

BR = 2560
GR = NP // BR

def _mesh():
    return plsc.VectorSubcoreMesh(core_axis_name="c", subcore_axis_name="s",
                                  num_cores=NC, num_subcores=NS)


def _tc1_body(x_ref, w1_ref, ams_ref, amd_ref,
              hc0_ref, hc1_ref, hc2_ref, hc3_ref,
              as_ref, ad_ref, gs_ref, gd_ref):
    i = pl.program_id(0)
    h = jnp.dot(x_ref[...], w1_ref[...], preferred_element_type=jnp.float32)
    hc0_ref[...] = h[:, 0:128]
    hc1_ref[...] = h[:, 128:256]
    hc2_ref[...] = h[:, 256:384]
    hc3_ref[...] = h[:, 384:512]
    asv = jnp.dot(h, ams_ref[...], preferred_element_type=jnp.float32)
    adv = jnp.dot(h, amd_ref[...], preferred_element_type=jnp.float32)
    row = i * BR + lax.broadcasted_iota(jnp.int32, (BR, 16), 0)
    asv = jnp.where(row < N, asv, NEG)
    adv = jnp.where(row < N, adv, NEG)
    as_ref[...] = asv
    ad_ref[...] = adv
    bs = jnp.max(asv, axis=0, keepdims=True)
    bd = jnp.max(adv, axis=0, keepdims=True)

    @pl.when(i == 0)
    def _():
        gs_ref[...] = jnp.broadcast_to(bs, (8, 16))
        gd_ref[...] = jnp.broadcast_to(bd, (8, 16))

    @pl.when(i > 0)
    def _():
        gs_ref[...] = jnp.maximum(gs_ref[...], bs)
        gd_ref[...] = jnp.maximum(gd_ref[...], bd)


def _tc1(xp, W1, ams, amd):
    f32 = jnp.float32
    blk = lambda shp: pl.BlockSpec(shp, lambda i: (i, 0))
    whole = lambda shp: pl.BlockSpec(shp, lambda i: (0, 0))
    return pl.pallas_call(
        _tc1_body,
        grid=(GR,),
        in_specs=[blk((BR, IN_DIM)), whole((IN_DIM, 512)),
                  whole((512, 16)), whole((512, 16))],
        out_specs=[blk((BR, 128)), blk((BR, 128)), blk((BR, 128)),
                   blk((BR, 128)), blk((BR, 16)), blk((BR, 16)),
                   whole((8, 16)), whole((8, 16))],
        out_shape=[jax.ShapeDtypeStruct((NP, 128), f32),
                   jax.ShapeDtypeStruct((NP, 128), f32),
                   jax.ShapeDtypeStruct((NP, 128), f32),
                   jax.ShapeDtypeStruct((NP, 128), f32),
                   jax.ShapeDtypeStruct((NP, 16), f32),
                   jax.ShapeDtypeStruct((NP, 16), f32),
                   jax.ShapeDtypeStruct((8, 16), f32),
                   jax.ShapeDtypeStruct((8, 16), f32)],
    )(xp, W1, ams, amd)


def _ln_elu(o, lw, lb):
    mu = jnp.mean(o, axis=1, keepdims=True)
    var = jnp.mean((o - mu) ** 2, axis=1, keepdims=True)
    o = (o - mu) / jnp.sqrt(var + EPS) * lw + lb
    return jnp.where(o > 0, o, jnp.exp(jnp.minimum(o, 0.0)) - 1.0)


def _tc2_body(oa_ref, ob_ref, oc_ref, od_ref, d0_ref, d1_ref, b1_ref, lw_ref, lb_ref,
              ex16_ref, w2_ref, a2s_ref, a2d_ref,
              h2_ref, as2_ref, ad2_ref, gs_ref, gd_ref):
    i = pl.program_id(0)
    den = d0_ref[...] + d1_ref[...]
    dinv = 1.0 / (den + 1e-16)
    dinv512 = jnp.dot(dinv, ex16_ref[...], preferred_element_type=jnp.float32)
    o = jnp.concatenate([oa_ref[...], ob_ref[...], oc_ref[...], od_ref[...]],
                        axis=1)
    o = o * dinv512 + b1_ref[...]
    o = _ln_elu(o, lw_ref[...], lb_ref[...])
    h2 = jnp.dot(o, w2_ref[...], preferred_element_type=jnp.float32)
    h2_ref[...] = h2
    asv = jnp.dot(h2, a2s_ref[...], preferred_element_type=jnp.float32)
    adv = jnp.dot(h2, a2d_ref[...], preferred_element_type=jnp.float32)
    row = i * BR + lax.broadcasted_iota(jnp.int32, (BR, 16), 0)
    asv = jnp.where(row < N, asv, NEG)
    adv = jnp.where(row < N, adv, NEG)
    as2_ref[...] = asv
    ad2_ref[...] = adv
    bs = jnp.max(asv, axis=0, keepdims=True)
    bd = jnp.max(adv, axis=0, keepdims=True)

    @pl.when(i == 0)
    def _():
        gs_ref[...] = jnp.broadcast_to(bs, (8, 16))
        gd_ref[...] = jnp.broadcast_to(bd, (8, 16))

    @pl.when(i > 0)
    def _():
        gs_ref[...] = jnp.maximum(gs_ref[...], bs)
        gd_ref[...] = jnp.maximum(gd_ref[...], bd)


def _tc2(oa, ob, oc, od, d0, d1, b1r, lw, lb, ex16, W2, a2s, a2d):
    f32 = jnp.float32
    blk = lambda shp: pl.BlockSpec(shp, lambda i: (i, 0))
    whole = lambda shp: pl.BlockSpec(shp, lambda i: (0, 0))
    return pl.pallas_call(
        _tc2_body,
        grid=(GR,),
        in_specs=[blk((BR, 128)), blk((BR, 128)), blk((BR, 128)), blk((BR, 128)),
                  blk((BR, 16)), blk((BR, 16)),
                  whole((1, 512)), whole((1, 512)), whole((1, 512)),
                  whole((16, 512)), whole((512, HID)),
                  whole((HID, 16)), whole((HID, 16))],
        out_specs=[blk((BR, HID)), blk((BR, 16)), blk((BR, 16)),
                   whole((8, 16)), whole((8, 16))],
        out_shape=[jax.ShapeDtypeStruct((NP, HID), f32),
                   jax.ShapeDtypeStruct((NP, 16), f32),
                   jax.ShapeDtypeStruct((NP, 16), f32),
                   jax.ShapeDtypeStruct((8, 16), f32),
                   jax.ShapeDtypeStruct((8, 16), f32)],
    )(oa, ob, oc, od, d0, d1, b1r, lw, lb, ex16, W2, a2s, a2d)


def _tc3_body(p0_ref, p1_ref, d0_ref, d1_ref, sel_ref, b2_ref, lw_ref, lb_ref,
              out_ref):
    den = d0_ref[...] + d1_ref[...]
    dinv = 1.0 / (den + 1e-16)
    dinv64 = jnp.dot(dinv, sel_ref[...], preferred_element_type=jnp.float32)
    o = (p0_ref[...] + p1_ref[...]) * dinv64 + b2_ref[...]
    out_ref[...] = _ln_elu(o, lw_ref[...], lb_ref[...])


def _tc3(p0, p1, d0, d1, sel, b2r, lw, lb):
    blk = lambda shp: pl.BlockSpec(shp, lambda i: (i, 0))
    whole = lambda shp: pl.BlockSpec(shp, lambda i: (0, 0))
    return pl.pallas_call(
        _tc3_body,
        grid=(GR,),
        in_specs=[blk((BR, HID)), blk((BR, HID)), blk((BR, 16)), blk((BR, 16)),
                  whole((16, HID)), whole((1, HID)), whole((1, HID)),
                  whole((1, HID))],
        out_specs=blk((BR, HID)),
        out_shape=jax.ShapeDtypeStruct((NP, HID), jnp.float32),
    )(p0, p1, d0, d1, sel, b2r, lw, lb)


def _att1_body(src_h, dst_h, as_h, ad_h, g_h, z_h,
               ex_h, den_h,
               den_sh, srcv, dstv, g1v, g2v, exv, gv, wbv):
    c = lax.axis_index("c")
    s = lax.axis_index("s")
    w = s * NC + c
    pltpu.sync_copy(z_h, den_sh.at[pl.ds(s * STRIPE, STRIPE)])
    pltpu.sync_copy(g_h, gv)
    plsc.subcore_barrier()
    gvec = gv[...]
    base = w * (EP // 32)

    @pl.loop(0, (EP // 32) // CH)
    def _chunk(t):
        eb = base + t * CH
        pltpu.sync_copy(src_h.at[pl.ds(eb, CH)], srcv)
        pltpu.sync_copy(dst_h.at[pl.ds(eb, CH)], dstv)
        pltpu.sync_copy(as_h.at[srcv], g1v)
        pltpu.sync_copy(ad_h.at[dstv], g2v)

        @pl.loop(0, CH)
        def _edge(j):
            a = g1v[j] + g2v[j]
            a = jnp.where(a >= 0.0, a, 0.2 * a)
            exv[j] = jnp.exp(a - gvec)

        pltpu.sync_copy(exv, ex_h.at[pl.ds(eb, CH)])
        pltpu.sync_copy(exv, den_sh.at[dstv], add=True)

    plsc.subcore_barrier()
    pltpu.sync_copy(den_sh.at[pl.ds(s * STRIPE, STRIPE)], wbv)
    pltpu.sync_copy(wbv, den_h.at[pl.ds(c * NP + s * STRIPE, STRIPE)])


def _att1(src, dst, as16, ad16, g16, z16):
    f32 = jnp.float32
    return pl.kernel(
        _att1_body,
        out_type=[jax.ShapeDtypeStruct((EP, 16), f32),
                  jax.ShapeDtypeStruct((2 * NP, 16), f32)],
        mesh=_mesh(),
        compiler_params=pltpu.CompilerParams(use_tc_tiling_on_sc=False),
        scratch_types=[
            pltpu.VMEM_SHARED((NP, 16), f32),
            pltpu.VMEM((CH,), jnp.int32),
            pltpu.VMEM((CH,), jnp.int32),
            pltpu.VMEM((CH, 16), f32),
            pltpu.VMEM((CH, 16), f32),
            pltpu.VMEM((CH, 16), f32),
            pltpu.VMEM((16,), f32),
            pltpu.VMEM((STRIPE, 16), f32),
        ],
    )(src, dst, as16, ad16, g16, z16)


CHM = 64


def _fpass128(src_h, dst_h, ex_h, h_h, o_h, out_sh, s, f, z_h,
              srcA, dstA, exA, rowsA, gsA, ssA,
              srcB, dstB, exB, rowsB, gsB, wbv):
    NCHT = (EP // 16) // CHM
    pltpu.sync_copy(z_h, out_sh.at[pl.ds(s * STRIPE, STRIPE)])
    plsc.subcore_barrier()
    ebase = s * (EP // 16)

    def load_idx(t, srcp, dstp):
        eb = ebase + t * CHM
        pltpu.sync_copy(src_h.at[pl.ds(eb, CHM)], srcp)
        pltpu.sync_copy(dst_h.at[pl.ds(eb, CHM)], dstp)

    def fire_gather(srcp, rowsp, gsp):
        pltpu.async_copy(h_h.at[srcp], rowsp, gsp)

    def wait_gather(srcp, rowsp, gsp):
        pltpu.make_async_copy(h_h.at[srcp], rowsp, gsp).wait()

    def scale(t, exp_, rowsp):
        pltpu.sync_copy(ex_h.at[pl.ds(ebase + t * CHM, CHM)], exp_)

        @pl.loop(0, CHM, unroll=2)
        def _edge(j):
            exrow = exp_[j, :]
            for k in range(8):
                sc = exrow[f * 2 + (k // 4)]
                rowsp[j, pl.ds(k * 16, 16)] = rowsp[j, pl.ds(k * 16, 16)] * sc

    load_idx(0, srcA, dstA)
    fire_gather(srcA, rowsA, gsA)

    @pl.loop(0, NCHT // 2)
    def _u(u):
        t0 = 2 * u
        t1 = 2 * u + 1
        t2 = 2 * u + 2

        load_idx(t1, srcB, dstB)
        fire_gather(srcB, rowsB, gsB)
        wait_gather(srcA, rowsA, gsA)
        scale(t0, exA, rowsA)
        pltpu.async_copy(rowsA, out_sh.at[dstA], ssA, add=True)

        @pl.when(t2 < NCHT)
        def _():
            pltpu.make_async_copy(rowsA, out_sh.at[dstA], ssA).wait()
            load_idx(t2, srcA, dstA)
            fire_gather(srcA, rowsA, gsA)

        wait_gather(srcB, rowsB, gsB)
        scale(t1, exB, rowsB)
        pltpu.sync_copy(rowsB, out_sh.at[dstB], add=True)

    pltpu.make_async_copy(rowsA, out_sh.at[dstA], ssA).wait()
    plsc.subcore_barrier()
    rb = s * STRIPE
    for q in range(STRIPE // 16):
        pltpu.sync_copy(out_sh.at[pl.ds(rb + q * 16, 16)], wbv)
        pltpu.sync_copy(wbv, o_h.at[pl.ds(rb + q * 16, 16)])
    plsc.subcore_barrier()


def _msg1_body(src_h, dst_h, ex_h, hc0_h, hc1_h, hc2_h, hc3_h, z_h,
               oc0_h, oc1_h, oc2_h, oc3_h,
               out_sh, srcA, dstA, exA, rowsA, srcB, dstB, exB, rowsB,
               gsA, ssA, gsB, wbv):
    c = lax.axis_index("c")
    s = lax.axis_index("s")
    hs = (hc0_h, hc1_h, hc2_h, hc3_h)
    os_ = (oc0_h, oc1_h, oc2_h, oc3_h)
    for cc in range(NC):
        @pl.when(c == cc)
        def _(cc=cc):
            for fp in range(2):
                f = cc * 2 + fp
                _fpass128(src_h, dst_h, ex_h, hs[f], os_[f], out_sh, s, f,
                          z_h, srcA, dstA, exA, rowsA, gsA, ssA,
                          srcB, dstB, exB, rowsB, gsB, wbv)


def _msg1(src, dst, ex1, hc0, hc1, hc2, hc3, z128):
    f32 = jnp.float32
    return pl.kernel(
        _msg1_body,
        out_type=[jax.ShapeDtypeStruct((NP, 128), f32)] * 4,
        mesh=_mesh(),
        compiler_params=pltpu.CompilerParams(use_tc_tiling_on_sc=False),
        scratch_types=[
            pltpu.VMEM_SHARED((NP, 128), f32),
            pltpu.VMEM((CHM,), jnp.int32),
            pltpu.VMEM((CHM,), jnp.int32),
            pltpu.VMEM((CHM, 16), f32),
            pltpu.VMEM((CHM, 128), f32),
            pltpu.VMEM((CHM,), jnp.int32),
            pltpu.VMEM((CHM,), jnp.int32),
            pltpu.VMEM((CHM, 16), f32),
            pltpu.VMEM((CHM, 128), f32),
            pltpu.SemaphoreType.DMA,
            pltpu.SemaphoreType.DMA,
            pltpu.SemaphoreType.DMA,
            pltpu.VMEM((16, 128), f32),
        ],
    )(src, dst, ex1, hc0, hc1, hc2, hc3, z128)


def _l2_body(src_h, dst_h, as_h, ad_h, g_h, h2_h, z16_h, z64_h,
             op_h, dp_h,
             out_sh, den_sh, srcv, dstv, g1v, g2v, exv, rows_v, gv, wbv):
    c = lax.axis_index("c")
    s = lax.axis_index("s")
    w = s * NC + c
    pltpu.sync_copy(z16_h, den_sh.at[pl.ds(s * STRIPE, STRIPE)])
    pltpu.sync_copy(z64_h, out_sh.at[pl.ds(s * STRIPE, STRIPE)])
    pltpu.sync_copy(g_h, gv)
    plsc.subcore_barrier()
    gvec = gv[...]
    base = w * (EP // 32)

    @pl.loop(0, (EP // 32) // CH)
    def _chunk(t):
        eb = base + t * CH
        pltpu.sync_copy(src_h.at[pl.ds(eb, CH)], srcv)
        pltpu.sync_copy(dst_h.at[pl.ds(eb, CH)], dstv)
        pltpu.sync_copy(as_h.at[srcv], g1v)
        pltpu.sync_copy(ad_h.at[dstv], g2v)

        @pl.loop(0, CH)
        def _edge(j):
            a = g1v[j] + g2v[j]
            a = jnp.where(a >= 0.0, a, 0.2 * a)
            exv[j] = jnp.exp(a - gvec)

        pltpu.sync_copy(exv, den_sh.at[dstv], add=True)
        pltpu.sync_copy(h2_h.at[srcv], rows_v)

        @pl.loop(0, CH)
        def _scale(j):
            sc = exv[j, :][0]
            for k in range(HID // 16):
                rows_v[j, pl.ds(k * 16, 16)] = rows_v[j, pl.ds(k * 16, 16)] * sc

        pltpu.sync_copy(rows_v, out_sh.at[dstv], add=True)

    plsc.subcore_barrier()
    pltpu.sync_copy(den_sh.at[pl.ds(s * STRIPE, STRIPE)], wbv)
    pltpu.sync_copy(wbv, dp_h.at[pl.ds(c * NP + s * STRIPE, STRIPE)])
    rb = s * STRIPE
    for off, sz in ((0, CH), (CH, CH), (2 * CH, CH), (3 * CH, CH),
                    (4 * CH, STRIPE - 4 * CH)):
        pltpu.sync_copy(out_sh.at[pl.ds(rb + off, sz)],
                        rows_v.at[pl.ds(0, sz)])
        pltpu.sync_copy(rows_v.at[pl.ds(0, sz)],
                        op_h.at[pl.ds(c * NP + rb + off, sz)])


def _l2(src, dst, as2, ad2, g16, h2, z16, z64):
    f32 = jnp.float32
    return pl.kernel(
        _l2_body,
        out_type=[jax.ShapeDtypeStruct((2 * NP, HID), f32),
                  jax.ShapeDtypeStruct((2 * NP, 16), f32)],
        mesh=_mesh(),
        compiler_params=pltpu.CompilerParams(use_tc_tiling_on_sc=False),
        scratch_types=[
            pltpu.VMEM_SHARED((NP, HID), f32),
            pltpu.VMEM_SHARED((NP, 16), f32),
            pltpu.VMEM((CH,), jnp.int32),
            pltpu.VMEM((CH,), jnp.int32),
            pltpu.VMEM((CH, 16), f32),
            pltpu.VMEM((CH, 16), f32),
            pltpu.VMEM((CH, 16), f32),
            pltpu.VMEM((CH, HID), f32),
            pltpu.VMEM((16,), f32),
            pltpu.VMEM((STRIPE, 16), f32),
        ],
    )(src, dst, as2, ad2, g16, h2, z16, z64)


def _blockdiag16(att):
    h, d = att.shape
    eye = jnp.eye(8, dtype=att.dtype)[:h]
    m = att[:, :, None] * eye[:, None, :]
    m = m.reshape(h * d, 8)
    if h * d < 512:
        m = jnp.pad(m, ((0, 512 - h * d), (0, 0)))
    return jnp.concatenate([m, m], axis=1)


def kernel(x, edge_index, W1, att_src1, att_dst1, b1, ln1_w, ln1_b,
           W2, att_src2, att_dst2, b2, ln2_w, ln2_b):
    f32 = jnp.float32
    src = edge_index[0].astype(jnp.int32)
    dst = edge_index[1].astype(jnp.int32)
    loop = jnp.arange(N, dtype=jnp.int32)
    pad = jnp.full((EP - ET,), SENT, jnp.int32)
    src = jnp.concatenate([src, loop, pad])
    dst = jnp.concatenate([dst, loop, pad])

    xp = jnp.pad(x, ((0, NP - N), (0, 0)))

    ams1 = _blockdiag16(att_src1.reshape(HEADS, HID))
    amd1 = _blockdiag16(att_dst1.reshape(HEADS, HID))
    hc0, hc1, hc2, hc3, as1, ad1, gs1, gd1 = _tc1(xp, W1, ams1, amd1)
    g1 = jnp.max(gs1, axis=0) + jnp.max(gd1, axis=0)

    z16 = jnp.zeros((STRIPE, 16), f32)
    z64 = jnp.zeros((STRIPE, HID), f32)
    z128 = jnp.zeros((STRIPE, 128), f32)

    ex1, den1 = _att1(src, dst, as1, ad1, g1, z16)
    oc0, oc1, oc2, oc3 = _msg1(src, dst, ex1, hc0, hc1, hc2, hc3, z128)

    ex16 = jnp.concatenate(
        [jnp.repeat(jnp.eye(8, dtype=f32), HID, axis=1),
         jnp.zeros((8, 512), f32)], axis=0)
    a2s = jnp.tile(att_src2.reshape(HID, 1), (1, 16))
    a2d = jnp.tile(att_dst2.reshape(HID, 1), (1, 16))
    h2, as2, ad2, gs2, gd2 = _tc2(
        oc0, oc1, oc2, oc3, den1[:NP], den1[NP:], b1.reshape(1, 512),
        ln1_w.reshape(1, 512), ln1_b.reshape(1, 512), ex16, W2, a2s, a2d)
    g2 = jnp.max(gs2, axis=0) + jnp.max(gd2, axis=0)

    op2, dp2 = _l2(src, dst, as2, ad2, g2, h2, z16, z64)

    sel = jnp.zeros((16, HID), f32).at[0, :].set(1.0)
    out = _tc3(op2[:NP], op2[NP:], dp2[:NP], dp2[NP:], sel,
               b2.reshape(1, HID), ln2_w.reshape(1, HID),
               ln2_b.reshape(1, HID))
    return out[:N]


# speedup vs baseline: 1.1706x; 1.1706x over previous
import jax
import jax.numpy as jnp
from jax import lax
from jax.experimental import pallas as pl
from jax.experimental.pallas import tpu as pltpu
from jax.experimental.pallas import tpu_sc as plsc

N = 10000
E = 320000
IN_DIM = 128
HID = 64
HEADS = 8
EPS = 1e-5

NP = 10240
ET = E + N
CH = 128
EP = 331776
SENT = NP - 1
NEG = -1e30

NC = 2
NS = 16
HALF = NP // 2
OB = HALF + 128
STRIPE = NP // NS

BR = 2560
GR = NP // BR

def _mesh():
    return plsc.VectorSubcoreMesh(core_axis_name="c", subcore_axis_name="s",
                                  num_cores=NC, num_subcores=NS)


def _tc1_body(x_ref, w1_ref, ams_ref, amd_ref,
              hc0_ref, hc1_ref, hc2_ref, hc3_ref,
              as_ref, ad_ref, gs_ref, gd_ref):
    i = pl.program_id(0)
    h = jnp.dot(x_ref[...], w1_ref[...], preferred_element_type=jnp.float32)
    hc0_ref[...] = h[:, 0:128]
    hc1_ref[...] = h[:, 128:256]
    hc2_ref[...] = h[:, 256:384]
    hc3_ref[...] = h[:, 384:512]
    asv = jnp.dot(h, ams_ref[...], preferred_element_type=jnp.float32)
    adv = jnp.dot(h, amd_ref[...], preferred_element_type=jnp.float32)
    row = i * BR + lax.broadcasted_iota(jnp.int32, (BR, 16), 0)
    asv = jnp.where(row < N, asv, NEG)
    adv = jnp.where(row < N, adv, NEG)
    as_ref[...] = asv
    ad_ref[...] = adv
    bs = jnp.max(asv, axis=0, keepdims=True)
    bd = jnp.max(adv, axis=0, keepdims=True)

    @pl.when(i == 0)
    def _():
        gs_ref[...] = jnp.broadcast_to(bs, (8, 16))
        gd_ref[...] = jnp.broadcast_to(bd, (8, 16))

    @pl.when(i > 0)
    def _():
        gs_ref[...] = jnp.maximum(gs_ref[...], bs)
        gd_ref[...] = jnp.maximum(gd_ref[...], bd)


def _tc1(xp, W1, ams, amd):
    f32 = jnp.float32
    blk = lambda shp: pl.BlockSpec(shp, lambda i: (i, 0))
    whole = lambda shp: pl.BlockSpec(shp, lambda i: (0, 0))
    return pl.pallas_call(
        _tc1_body,
        grid=(GR,),
        in_specs=[blk((BR, IN_DIM)), whole((IN_DIM, 512)),
                  whole((512, 16)), whole((512, 16))],
        out_specs=[blk((BR, 128)), blk((BR, 128)), blk((BR, 128)),
                   blk((BR, 128)), blk((BR, 16)), blk((BR, 16)),
                   whole((8, 16)), whole((8, 16))],
        out_shape=[jax.ShapeDtypeStruct((NP, 128), f32),
                   jax.ShapeDtypeStruct((NP, 128), f32),
                   jax.ShapeDtypeStruct((NP, 128), f32),
                   jax.ShapeDtypeStruct((NP, 128), f32),
                   jax.ShapeDtypeStruct((NP, 16), f32),
                   jax.ShapeDtypeStruct((NP, 16), f32),
                   jax.ShapeDtypeStruct((8, 16), f32),
                   jax.ShapeDtypeStruct((8, 16), f32)],
    )(xp, W1, ams, amd)


def _ln_elu(o, lw, lb):
    mu = jnp.mean(o, axis=1, keepdims=True)
    var = jnp.mean((o - mu) ** 2, axis=1, keepdims=True)
    o = (o - mu) / jnp.sqrt(var + EPS) * lw + lb
    return jnp.where(o > 0, o, jnp.exp(jnp.minimum(o, 0.0)) - 1.0)


def _tc2_body(oa_ref, ob_ref, oc_ref, od_ref, d0_ref, d1_ref, b1_ref, lw_ref, lb_ref,
              ex16_ref, w2_ref, a2s_ref, a2d_ref,
              h2_ref, as2_ref, ad2_ref, gs_ref, gd_ref):
    i = pl.program_id(0)
    den = d0_ref[...] + d1_ref[...]
    dinv = 1.0 / (den + 1e-16)
    dinv512 = jnp.dot(dinv, ex16_ref[...], preferred_element_type=jnp.float32)
    o = jnp.concatenate([oa_ref[...], ob_ref[...], oc_ref[...], od_ref[...]],
                        axis=1)
    o = o * dinv512 + b1_ref[...]
    o = _ln_elu(o, lw_ref[...], lb_ref[...])
    h2 = jnp.dot(o, w2_ref[...], preferred_element_type=jnp.float32)
    h2_ref[...] = h2
    asv = jnp.dot(h2, a2s_ref[...], preferred_element_type=jnp.float32)
    adv = jnp.dot(h2, a2d_ref[...], preferred_element_type=jnp.float32)
    row = i * BR + lax.broadcasted_iota(jnp.int32, (BR, 16), 0)
    asv = jnp.where(row < N, asv, NEG)
    adv = jnp.where(row < N, adv, NEG)
    as2_ref[...] = asv
    ad2_ref[...] = adv
    bs = jnp.max(asv, axis=0, keepdims=True)
    bd = jnp.max(adv, axis=0, keepdims=True)

    @pl.when(i == 0)
    def _():
        gs_ref[...] = jnp.broadcast_to(bs, (8, 16))
        gd_ref[...] = jnp.broadcast_to(bd, (8, 16))

    @pl.when(i > 0)
    def _():
        gs_ref[...] = jnp.maximum(gs_ref[...], bs)
        gd_ref[...] = jnp.maximum(gd_ref[...], bd)


def _tc2(oa, ob, oc, od, d0, d1, b1r, lw, lb, ex16, W2, a2s, a2d):
    f32 = jnp.float32
    blk = lambda shp: pl.BlockSpec(shp, lambda i: (i, 0))
    whole = lambda shp: pl.BlockSpec(shp, lambda i: (0, 0))
    return pl.pallas_call(
        _tc2_body,
        grid=(GR,),
        in_specs=[blk((BR, 128)), blk((BR, 128)), blk((BR, 128)), blk((BR, 128)),
                  blk((BR, 16)), blk((BR, 16)),
                  whole((1, 512)), whole((1, 512)), whole((1, 512)),
                  whole((16, 512)), whole((512, HID)),
                  whole((HID, 16)), whole((HID, 16))],
        out_specs=[blk((BR, HID)), blk((BR, 16)), blk((BR, 16)),
                   whole((8, 16)), whole((8, 16))],
        out_shape=[jax.ShapeDtypeStruct((NP, HID), f32),
                   jax.ShapeDtypeStruct((NP, 16), f32),
                   jax.ShapeDtypeStruct((NP, 16), f32),
                   jax.ShapeDtypeStruct((8, 16), f32),
                   jax.ShapeDtypeStruct((8, 16), f32)],
    )(oa, ob, oc, od, d0, d1, b1r, lw, lb, ex16, W2, a2s, a2d)


def _tc3_body(p0_ref, p1_ref, d0_ref, d1_ref, sel_ref, b2_ref, lw_ref, lb_ref,
              out_ref):
    den = d0_ref[...] + d1_ref[...]
    dinv = 1.0 / (den + 1e-16)
    dinv64 = jnp.dot(dinv, sel_ref[...], preferred_element_type=jnp.float32)
    o = (p0_ref[...] + p1_ref[...]) * dinv64 + b2_ref[...]
    out_ref[...] = _ln_elu(o, lw_ref[...], lb_ref[...])


def _tc3(p0, p1, d0, d1, sel, b2r, lw, lb):
    blk = lambda shp: pl.BlockSpec(shp, lambda i: (i, 0))
    whole = lambda shp: pl.BlockSpec(shp, lambda i: (0, 0))
    return pl.pallas_call(
        _tc3_body,
        grid=(GR,),
        in_specs=[blk((BR, HID)), blk((BR, HID)), blk((BR, 16)), blk((BR, 16)),
                  whole((16, HID)), whole((1, HID)), whole((1, HID)),
                  whole((1, HID))],
        out_specs=blk((BR, HID)),
        out_shape=jax.ShapeDtypeStruct((NP, HID), jnp.float32),
    )(p0, p1, d0, d1, sel, b2r, lw, lb)


def _att1_body(src_h, dst_h, as_h, ad_h, g_h, z_h,
               ex_h, den_h,
               den_sh, srcv, dstv, g1v, g2v, exv, gv, wbv):
    c = lax.axis_index("c")
    s = lax.axis_index("s")
    w = s * NC + c
    pltpu.sync_copy(z_h, den_sh.at[pl.ds(s * STRIPE, STRIPE)])
    pltpu.sync_copy(g_h, gv)
    plsc.subcore_barrier()
    gvec = gv[...]
    base = w * (EP // 32)

    @pl.loop(0, (EP // 32) // CH)
    def _chunk(t):
        eb = base + t * CH
        pltpu.sync_copy(src_h.at[pl.ds(eb, CH)], srcv)
        pltpu.sync_copy(dst_h.at[pl.ds(eb, CH)], dstv)
        pltpu.sync_copy(as_h.at[srcv], g1v)
        pltpu.sync_copy(ad_h.at[dstv], g2v)

        @pl.loop(0, CH)
        def _edge(j):
            a = g1v[j] + g2v[j]
            a = jnp.where(a >= 0.0, a, 0.2 * a)
            exv[j] = jnp.exp(a - gvec)

        pltpu.sync_copy(exv, ex_h.at[pl.ds(eb, CH)])
        pltpu.sync_copy(exv, den_sh.at[dstv], add=True)

    plsc.subcore_barrier()
    pltpu.sync_copy(den_sh.at[pl.ds(s * STRIPE, STRIPE)], wbv)
    pltpu.sync_copy(wbv, den_h.at[pl.ds(c * NP + s * STRIPE, STRIPE)])


def _att1(src, dst, as16, ad16, g16, z16):
    f32 = jnp.float32
    return pl.kernel(
        _att1_body,
        out_type=[jax.ShapeDtypeStruct((EP, 16), f32),
                  jax.ShapeDtypeStruct((2 * NP, 16), f32)],
        mesh=_mesh(),
        compiler_params=pltpu.CompilerParams(use_tc_tiling_on_sc=False),
        scratch_types=[
            pltpu.VMEM_SHARED((NP, 16), f32),
            pltpu.VMEM((CH,), jnp.int32),
            pltpu.VMEM((CH,), jnp.int32),
            pltpu.VMEM((CH, 16), f32),
            pltpu.VMEM((CH, 16), f32),
            pltpu.VMEM((CH, 16), f32),
            pltpu.VMEM((16,), f32),
            pltpu.VMEM((STRIPE, 16), f32),
        ],
    )(src, dst, as16, ad16, g16, z16)


CHM = 128


def _fpass128(src_h, dst_h, ex_h, h_h, o_h, out_sh, s, f, z_h,
              srcA, dstA, exA, rowsA, gsA, ssA,
              srcB, dstB, exB, rowsB, gsB, wbv):
    NCHT = (EP // 16) // CHM
    pltpu.sync_copy(z_h, out_sh.at[pl.ds(s * STRIPE, STRIPE)])
    plsc.subcore_barrier()
    ebase = s * (EP // 16)

    def load_idx(t, srcp, dstp):
        eb = ebase + t * CHM
        pltpu.sync_copy(src_h.at[pl.ds(eb, CHM)], srcp)
        pltpu.sync_copy(dst_h.at[pl.ds(eb, CHM)], dstp)

    def fire_gather(srcp, rowsp, gsp):
        pltpu.async_copy(h_h.at[srcp], rowsp, gsp)

    def wait_gather(srcp, rowsp, gsp):
        pltpu.make_async_copy(h_h.at[srcp], rowsp, gsp).wait()

    def scale(t, exp_, rowsp):
        pltpu.sync_copy(ex_h.at[pl.ds(ebase + t * CHM, CHM)], exp_)

        @pl.loop(0, CHM, unroll=2)
        def _edge(j):
            exrow = exp_[j, :]
            for k in range(8):
                sc = exrow[f * 2 + (k // 4)]
                rowsp[j, pl.ds(k * 16, 16)] = rowsp[j, pl.ds(k * 16, 16)] * sc

    load_idx(0, srcA, dstA)
    fire_gather(srcA, rowsA, gsA)

    @pl.loop(0, NCHT // 2)
    def _u(u):
        t0 = 2 * u
        t1 = 2 * u + 1
        t2 = 2 * u + 2

        load_idx(t1, srcB, dstB)
        fire_gather(srcB, rowsB, gsB)
        wait_gather(srcA, rowsA, gsA)
        scale(t0, exA, rowsA)
        pltpu.async_copy(rowsA, out_sh.at[dstA], ssA, add=True)

        @pl.when(t2 < NCHT)
        def _():
            pltpu.make_async_copy(rowsA, out_sh.at[dstA], ssA).wait()
            load_idx(t2, srcA, dstA)
            fire_gather(srcA, rowsA, gsA)

        wait_gather(srcB, rowsB, gsB)
        scale(t1, exB, rowsB)
        pltpu.sync_copy(rowsB, out_sh.at[dstB], add=True)

    pltpu.make_async_copy(rowsA, out_sh.at[dstA], ssA).wait()
    plsc.subcore_barrier()
    rb = s * STRIPE
    for q in range(STRIPE // 16):
        pltpu.sync_copy(out_sh.at[pl.ds(rb + q * 16, 16)], wbv)
        pltpu.sync_copy(wbv, o_h.at[pl.ds(rb + q * 16, 16)])
    plsc.subcore_barrier()


def _msg1_body(src_h, dst_h, ex_h, hc0_h, hc1_h, hc2_h, hc3_h, z_h,
               oc0_h, oc1_h, oc2_h, oc3_h,
               out_sh, srcA, dstA, exA, rowsA, srcB, dstB, exB, rowsB,
               gsA, ssA, gsB, wbv):
    c = lax.axis_index("c")
    s = lax.axis_index("s")
    hs = (hc0_h, hc1_h, hc2_h, hc3_h)
    os_ = (oc0_h, oc1_h, oc2_h, oc3_h)
    for cc in range(NC):
        @pl.when(c == cc)
        def _(cc=cc):
            for fp in range(2):
                f = cc * 2 + fp
                _fpass128(src_h, dst_h, ex_h, hs[f], os_[f], out_sh, s, f,
                          z_h, srcA, dstA, exA, rowsA, gsA, ssA,
                          srcB, dstB, exB, rowsB, gsB, wbv)


def _msg1(src, dst, ex1, hc0, hc1, hc2, hc3, z128):
    f32 = jnp.float32
    return pl.kernel(
        _msg1_body,
        out_type=[jax.ShapeDtypeStruct((NP, 128), f32)] * 4,
        mesh=_mesh(),
        compiler_params=pltpu.CompilerParams(use_tc_tiling_on_sc=False),
        scratch_types=[
            pltpu.VMEM_SHARED((NP, 128), f32),
            pltpu.VMEM((CHM,), jnp.int32),
            pltpu.VMEM((CHM,), jnp.int32),
            pltpu.VMEM((CHM, 16), f32),
            pltpu.VMEM((CHM, 128), f32),
            pltpu.VMEM((CHM,), jnp.int32),
            pltpu.VMEM((CHM,), jnp.int32),
            pltpu.VMEM((CHM, 16), f32),
            pltpu.VMEM((CHM, 128), f32),
            pltpu.SemaphoreType.DMA,
            pltpu.SemaphoreType.DMA,
            pltpu.SemaphoreType.DMA,
            pltpu.VMEM((16, 128), f32),
        ],
    )(src, dst, ex1, hc0, hc1, hc2, hc3, z128)


def _l2_body(src_h, dst_h, as_h, ad_h, g_h, h2_h, z16_h, z64_h,
             op_h, dp_h,
             out_sh, den_sh, srcv, dstv, g1v, g2v, exv, rows_v, gv, wbv):
    c = lax.axis_index("c")
    s = lax.axis_index("s")
    w = s * NC + c
    pltpu.sync_copy(z16_h, den_sh.at[pl.ds(s * STRIPE, STRIPE)])
    pltpu.sync_copy(z64_h, out_sh.at[pl.ds(s * STRIPE, STRIPE)])
    pltpu.sync_copy(g_h, gv)
    plsc.subcore_barrier()
    gvec = gv[...]
    base = w * (EP // 32)

    @pl.loop(0, (EP // 32) // CH)
    def _chunk(t):
        eb = base + t * CH
        pltpu.sync_copy(src_h.at[pl.ds(eb, CH)], srcv)
        pltpu.sync_copy(dst_h.at[pl.ds(eb, CH)], dstv)
        pltpu.sync_copy(as_h.at[srcv], g1v)
        pltpu.sync_copy(ad_h.at[dstv], g2v)

        @pl.loop(0, CH)
        def _edge(j):
            a = g1v[j] + g2v[j]
            a = jnp.where(a >= 0.0, a, 0.2 * a)
            exv[j] = jnp.exp(a - gvec)

        pltpu.sync_copy(exv, den_sh.at[dstv], add=True)
        pltpu.sync_copy(h2_h.at[srcv], rows_v)

        @pl.loop(0, CH)
        def _scale(j):
            sc = exv[j, :][0]
            for k in range(HID // 16):
                rows_v[j, pl.ds(k * 16, 16)] = rows_v[j, pl.ds(k * 16, 16)] * sc

        pltpu.sync_copy(rows_v, out_sh.at[dstv], add=True)

    plsc.subcore_barrier()
    pltpu.sync_copy(den_sh.at[pl.ds(s * STRIPE, STRIPE)], wbv)
    pltpu.sync_copy(wbv, dp_h.at[pl.ds(c * NP + s * STRIPE, STRIPE)])
    rb = s * STRIPE
    for off, sz in ((0, CH), (CH, CH), (2 * CH, CH), (3 * CH, CH),
                    (4 * CH, STRIPE - 4 * CH)):
        pltpu.sync_copy(out_sh.at[pl.ds(rb + off, sz)],
                        rows_v.at[pl.ds(0, sz)])
        pltpu.sync_copy(rows_v.at[pl.ds(0, sz)],
                        op_h.at[pl.ds(c * NP + rb + off, sz)])


def _l2(src, dst, as2, ad2, g16, h2, z16, z64):
    f32 = jnp.float32
    return pl.kernel(
        _l2_body,
        out_type=[jax.ShapeDtypeStruct((2 * NP, HID), f32),
                  jax.ShapeDtypeStruct((2 * NP, 16), f32)],
        mesh=_mesh(),
        compiler_params=pltpu.CompilerParams(use_tc_tiling_on_sc=False),
        scratch_types=[
            pltpu.VMEM_SHARED((NP, HID), f32),
            pltpu.VMEM_SHARED((NP, 16), f32),
            pltpu.VMEM((CH,), jnp.int32),
            pltpu.VMEM((CH,), jnp.int32),
            pltpu.VMEM((CH, 16), f32),
            pltpu.VMEM((CH, 16), f32),
            pltpu.VMEM((CH, 16), f32),
            pltpu.VMEM((CH, HID), f32),
            pltpu.VMEM((16,), f32),
            pltpu.VMEM((STRIPE, 16), f32),
        ],
    )(src, dst, as2, ad2, g16, h2, z16, z64)


def _blockdiag16(att):
    h, d = att.shape
    eye = jnp.eye(8, dtype=att.dtype)[:h]
    m = att[:, :, None] * eye[:, None, :]
    m = m.reshape(h * d, 8)
    if h * d < 512:
        m = jnp.pad(m, ((0, 512 - h * d), (0, 0)))
    return jnp.concatenate([m, m], axis=1)


def kernel(x, edge_index, W1, att_src1, att_dst1, b1, ln1_w, ln1_b,
           W2, att_src2, att_dst2, b2, ln2_w, ln2_b):
    f32 = jnp.float32
    src = edge_index[0].astype(jnp.int32)
    dst = edge_index[1].astype(jnp.int32)
    loop = jnp.arange(N, dtype=jnp.int32)
    pad = jnp.full((EP - ET,), SENT, jnp.int32)
    src = jnp.concatenate([src, loop, pad])
    dst = jnp.concatenate([dst, loop, pad])

    xp = jnp.pad(x, ((0, NP - N), (0, 0)))

    ams1 = _blockdiag16(att_src1.reshape(HEADS, HID))
    amd1 = _blockdiag16(att_dst1.reshape(HEADS, HID))
    hc0, hc1, hc2, hc3, as1, ad1, gs1, gd1 = _tc1(xp, W1, ams1, amd1)
    g1 = jnp.max(gs1, axis=0) + jnp.max(gd1, axis=0)

    z16 = jnp.zeros((STRIPE, 16), f32)
    z64 = jnp.zeros((STRIPE, HID), f32)
    z128 = jnp.zeros((STRIPE, 128), f32)

    ex1, den1 = _att1(src, dst, as1, ad1, g1, z16)
    oc0, oc1, oc2, oc3 = _msg1(src, dst, ex1, hc0, hc1, hc2, hc3, z128)

    ex16 = jnp.concatenate(
        [jnp.repeat(jnp.eye(8, dtype=f32), HID, axis=1),
         jnp.zeros((8, 512), f32)], axis=0)
    a2s = jnp.tile(att_src2.reshape(HID, 1), (1, 16))
    a2d = jnp.tile(att_dst2.reshape(HID, 1), (1, 16))
    h2, as2, ad2, gs2, gd2 = _tc2(
        oc0, oc1, oc2, oc3, den1[:NP], den1[NP:], b1.reshape(1, 512),
        ln1_w.reshape(1, 512), ln1_b.reshape(1, 512), ex16, W2, a2s, a2d)
    g2 = jnp.max(gs2, axis=0) + jnp.max(gd2, axis=0)

    op2, dp2 = _l2(src, dst, as2, ad2, g2, h2, z16, z64)

    sel = jnp.zeros((16, HID), f32).at[0, :].set(1.0)
    out = _tc3(op2[:NP], op2[NP:], dp2[:NP], dp2[NP:], sel,
               b2.reshape(1, HID), ln2_w.reshape(1, HID),
               ln2_b.reshape(1, HID))
    return out[:N]


# speedup vs baseline: 1.3333x; 1.1390x over previous
import jax
import jax.numpy as jnp
from jax import lax
from jax.experimental import pallas as pl
from jax.experimental.pallas import tpu as pltpu
from jax.experimental.pallas import tpu_sc as plsc

N = 10000
E = 320000
IN_DIM = 128
HID = 64
HEADS = 8
EPS = 1e-5

NP = 10240
ET = E + N
CH = 128
EP = 331776
SENT = NP - 1
NEG = -1e30

NC = 2
NS = 16
HALF = NP // 2
OB = HALF + 128
STRIPE = NP // NS

BR = 2560
GR = NP // BR

def _mesh():
    return plsc.VectorSubcoreMesh(core_axis_name="c", subcore_axis_name="s",
                                  num_cores=NC, num_subcores=NS)


def _tc1_body(x_ref, w1_ref, ams_ref, amd_ref,
              hc0_ref, hc1_ref, hc2_ref, hc3_ref,
              as_ref, ad_ref, gs_ref, gd_ref):
    i = pl.program_id(0)
    h = jnp.dot(x_ref[...], w1_ref[...], preferred_element_type=jnp.float32)
    hc0_ref[...] = h[:, 0:128]
    hc1_ref[...] = h[:, 128:256]
    hc2_ref[...] = h[:, 256:384]
    hc3_ref[...] = h[:, 384:512]
    asv = jnp.dot(h, ams_ref[...], preferred_element_type=jnp.float32)
    adv = jnp.dot(h, amd_ref[...], preferred_element_type=jnp.float32)
    row = i * BR + lax.broadcasted_iota(jnp.int32, (BR, 16), 0)
    asv = jnp.where(row < N, asv, NEG)
    adv = jnp.where(row < N, adv, NEG)
    as_ref[...] = asv
    ad_ref[...] = adv
    bs = jnp.max(asv, axis=0, keepdims=True)
    bd = jnp.max(adv, axis=0, keepdims=True)

    @pl.when(i == 0)
    def _():
        gs_ref[...] = jnp.broadcast_to(bs, (8, 16))
        gd_ref[...] = jnp.broadcast_to(bd, (8, 16))

    @pl.when(i > 0)
    def _():
        gs_ref[...] = jnp.maximum(gs_ref[...], bs)
        gd_ref[...] = jnp.maximum(gd_ref[...], bd)


def _tc1(xp, W1, ams, amd):
    f32 = jnp.float32
    blk = lambda shp: pl.BlockSpec(shp, lambda i: (i, 0))
    whole = lambda shp: pl.BlockSpec(shp, lambda i: (0, 0))
    return pl.pallas_call(
        _tc1_body,
        grid=(GR,),
        in_specs=[blk((BR, IN_DIM)), whole((IN_DIM, 512)),
                  whole((512, 16)), whole((512, 16))],
        out_specs=[blk((BR, 128)), blk((BR, 128)), blk((BR, 128)),
                   blk((BR, 128)), blk((BR, 16)), blk((BR, 16)),
                   whole((8, 16)), whole((8, 16))],
        out_shape=[jax.ShapeDtypeStruct((NP, 128), f32),
                   jax.ShapeDtypeStruct((NP, 128), f32),
                   jax.ShapeDtypeStruct((NP, 128), f32),
                   jax.ShapeDtypeStruct((NP, 128), f32),
                   jax.ShapeDtypeStruct((NP, 16), f32),
                   jax.ShapeDtypeStruct((NP, 16), f32),
                   jax.ShapeDtypeStruct((8, 16), f32),
                   jax.ShapeDtypeStruct((8, 16), f32)],
    )(xp, W1, ams, amd)


def _ln_elu(o, lw, lb):
    mu = jnp.mean(o, axis=1, keepdims=True)
    var = jnp.mean((o - mu) ** 2, axis=1, keepdims=True)
    o = (o - mu) / jnp.sqrt(var + EPS) * lw + lb
    return jnp.where(o > 0, o, jnp.exp(jnp.minimum(o, 0.0)) - 1.0)


def _tc2_body(oa_ref, ob_ref, oc_ref, od_ref, d0_ref, d1_ref, b1_ref, lw_ref, lb_ref,
              ex16_ref, w2_ref, a2s_ref, a2d_ref,
              h2_ref, as2_ref, ad2_ref, gs_ref, gd_ref):
    i = pl.program_id(0)
    den = d0_ref[...] + d1_ref[...]
    dinv = 1.0 / (den + 1e-16)
    dinv512 = jnp.dot(dinv, ex16_ref[...], preferred_element_type=jnp.float32)
    o = jnp.concatenate([oa_ref[...], ob_ref[...], oc_ref[...], od_ref[...]],
                        axis=1)
    o = o * dinv512 + b1_ref[...]
    o = _ln_elu(o, lw_ref[...], lb_ref[...])
    h2 = jnp.dot(o, w2_ref[...], preferred_element_type=jnp.float32)
    h2_ref[...] = h2
    asv = jnp.dot(h2, a2s_ref[...], preferred_element_type=jnp.float32)
    adv = jnp.dot(h2, a2d_ref[...], preferred_element_type=jnp.float32)
    row = i * BR + lax.broadcasted_iota(jnp.int32, (BR, 16), 0)
    asv = jnp.where(row < N, asv, NEG)
    adv = jnp.where(row < N, adv, NEG)
    as2_ref[...] = asv
    ad2_ref[...] = adv
    bs = jnp.max(asv, axis=0, keepdims=True)
    bd = jnp.max(adv, axis=0, keepdims=True)

    @pl.when(i == 0)
    def _():
        gs_ref[...] = jnp.broadcast_to(bs, (8, 16))
        gd_ref[...] = jnp.broadcast_to(bd, (8, 16))

    @pl.when(i > 0)
    def _():
        gs_ref[...] = jnp.maximum(gs_ref[...], bs)
        gd_ref[...] = jnp.maximum(gd_ref[...], bd)


def _tc2(oa, ob, oc, od, d0, d1, b1r, lw, lb, ex16, W2, a2s, a2d):
    f32 = jnp.float32
    blk = lambda shp: pl.BlockSpec(shp, lambda i: (i, 0))
    whole = lambda shp: pl.BlockSpec(shp, lambda i: (0, 0))
    return pl.pallas_call(
        _tc2_body,
        grid=(GR,),
        in_specs=[blk((BR, 128)), blk((BR, 128)), blk((BR, 128)), blk((BR, 128)),
                  blk((BR, 16)), blk((BR, 16)),
                  whole((1, 512)), whole((1, 512)), whole((1, 512)),
                  whole((16, 512)), whole((512, HID)),
                  whole((HID, 16)), whole((HID, 16))],
        out_specs=[blk((BR, HID)), blk((BR, 16)), blk((BR, 16)),
                   whole((8, 16)), whole((8, 16))],
        out_shape=[jax.ShapeDtypeStruct((NP, HID), f32),
                   jax.ShapeDtypeStruct((NP, 16), f32),
                   jax.ShapeDtypeStruct((NP, 16), f32),
                   jax.ShapeDtypeStruct((8, 16), f32),
                   jax.ShapeDtypeStruct((8, 16), f32)],
    )(oa, ob, oc, od, d0, d1, b1r, lw, lb, ex16, W2, a2s, a2d)


def _tc3_body(p0_ref, p1_ref, d0_ref, d1_ref, sel_ref, b2_ref, lw_ref, lb_ref,
              out_ref):
    den = d0_ref[...] + d1_ref[...]
    dinv = 1.0 / (den + 1e-16)
    dinv64 = jnp.dot(dinv, sel_ref[...], preferred_element_type=jnp.float32)
    o = (p0_ref[...] + p1_ref[...]) * dinv64 + b2_ref[...]
    out_ref[...] = _ln_elu(o, lw_ref[...], lb_ref[...])


def _tc3(p0, p1, d0, d1, sel, b2r, lw, lb):
    blk = lambda shp: pl.BlockSpec(shp, lambda i: (i, 0))
    whole = lambda shp: pl.BlockSpec(shp, lambda i: (0, 0))
    return pl.pallas_call(
        _tc3_body,
        grid=(GR,),
        in_specs=[blk((BR, HID)), blk((BR, HID)), blk((BR, 16)), blk((BR, 16)),
                  whole((16, HID)), whole((1, HID)), whole((1, HID)),
                  whole((1, HID))],
        out_specs=blk((BR, HID)),
        out_shape=jax.ShapeDtypeStruct((NP, HID), jnp.float32),
    )(p0, p1, d0, d1, sel, b2r, lw, lb)


def _att1_body(src_h, dst_h, as_h, ad_h, g_h, z_h,
               ex_h, den_h,
               den_sh, srcv, dstv, g1v, g2v, exv, gv, wbv):
    c = lax.axis_index("c")
    s = lax.axis_index("s")
    w = s * NC + c
    pltpu.sync_copy(z_h, den_sh.at[pl.ds(s * STRIPE, STRIPE)])
    pltpu.sync_copy(g_h, gv)
    plsc.subcore_barrier()
    gvec = gv[...]
    base = w * (EP // 32)

    @pl.loop(0, (EP // 32) // CH)
    def _chunk(t):
        eb = base + t * CH
        pltpu.sync_copy(src_h.at[pl.ds(eb, CH)], srcv)
        pltpu.sync_copy(dst_h.at[pl.ds(eb, CH)], dstv)
        pltpu.sync_copy(as_h.at[srcv], g1v)
        pltpu.sync_copy(ad_h.at[dstv], g2v)

        @pl.loop(0, CH)
        def _edge(j):
            a = g1v[j] + g2v[j]
            a = jnp.where(a >= 0.0, a, 0.2 * a)
            exv[j] = jnp.exp(a - gvec)

        pltpu.sync_copy(exv, ex_h.at[pl.ds(eb, CH)])
        pltpu.sync_copy(exv, den_sh.at[dstv], add=True)

    plsc.subcore_barrier()
    pltpu.sync_copy(den_sh.at[pl.ds(s * STRIPE, STRIPE)], wbv)
    pltpu.sync_copy(wbv, den_h.at[pl.ds(c * NP + s * STRIPE, STRIPE)])


def _att1(src, dst, as16, ad16, g16, z16):
    f32 = jnp.float32
    return pl.kernel(
        _att1_body,
        out_type=[jax.ShapeDtypeStruct((EP, 16), f32),
                  jax.ShapeDtypeStruct((2 * NP, 16), f32)],
        mesh=_mesh(),
        compiler_params=pltpu.CompilerParams(use_tc_tiling_on_sc=False),
        scratch_types=[
            pltpu.VMEM_SHARED((NP, 16), f32),
            pltpu.VMEM((CH,), jnp.int32),
            pltpu.VMEM((CH,), jnp.int32),
            pltpu.VMEM((CH, 16), f32),
            pltpu.VMEM((CH, 16), f32),
            pltpu.VMEM((CH, 16), f32),
            pltpu.VMEM((16,), f32),
            pltpu.VMEM((STRIPE, 16), f32),
        ],
    )(src, dst, as16, ad16, g16, z16)


CHM = 128
SG = 6


def _fpass128(src2_h, dst2_h, ex_h, h_h, o_h, out_sh, s, f, z_h,
              srcv2, dstv2, exv6, rowsA, gsA, ssA, rowsB, gsB, wbv):
    NCHT = (EP // 16) // CHM
    pltpu.sync_copy(z_h, out_sh.at[pl.ds(s * STRIPE, STRIPE)])
    plsc.subcore_barrier()
    ebase = s * (EP // 16)
    crow = s * NCHT

    def scale(t, rowsp):
        @pl.loop(0, CHM, unroll=2)
        def _edge(j):
            exrow = exv6[t * CHM + j, :]
            for k in range(8):
                sc = exrow[f * 2 + (k // 4)]
                rowsp[j, pl.ds(k * 16, 16)] = rowsp[j, pl.ds(k * 16, 16)] * sc

    @pl.loop(0, NCHT // SG)
    def _u(u):
        pltpu.sync_copy(src2_h.at[pl.ds(crow + u * SG, SG)], srcv2)
        pltpu.sync_copy(dst2_h.at[pl.ds(crow + u * SG, SG)], dstv2)
        pltpu.sync_copy(ex_h.at[pl.ds(ebase + u * SG * CHM, SG * CHM)], exv6)
        pltpu.async_copy(h_h.at[srcv2.at[0]], rowsA, gsA)
        for p in range(SG // 2):
            tA, tB = 2 * p, 2 * p + 1
            pltpu.async_copy(h_h.at[srcv2.at[tB]], rowsB, gsB)
            pltpu.make_async_copy(h_h.at[srcv2.at[tA]], rowsA, gsA).wait()
            scale(tA, rowsA)
            pltpu.async_copy(rowsA, out_sh.at[dstv2.at[tA]], ssA, add=True)
            if p < SG // 2 - 1:
                pltpu.make_async_copy(rowsA, out_sh.at[dstv2.at[tA]],
                                      ssA).wait()
                pltpu.async_copy(h_h.at[srcv2.at[tA + 2]], rowsA, gsA)
            pltpu.make_async_copy(h_h.at[srcv2.at[tB]], rowsB, gsB).wait()
            scale(tB, rowsB)
            pltpu.sync_copy(rowsB, out_sh.at[dstv2.at[tB]], add=True)
        pltpu.make_async_copy(rowsA, out_sh.at[dstv2.at[SG - 2]], ssA).wait()

    plsc.subcore_barrier()
    rb = s * STRIPE
    for q in range(STRIPE // 16):
        pltpu.sync_copy(out_sh.at[pl.ds(rb + q * 16, 16)], wbv)
        pltpu.sync_copy(wbv, o_h.at[pl.ds(rb + q * 16, 16)])
    plsc.subcore_barrier()


def _msg1_body(src2_h, dst2_h, ex_h, hc0_h, hc1_h, hc2_h, hc3_h, z_h,
               oc0_h, oc1_h, oc2_h, oc3_h,
               out_sh, srcv2, dstv2, exv6, rowsA, rowsB,
               gsA, ssA, gsB, wbv):
    c = lax.axis_index("c")
    s = lax.axis_index("s")
    hs = (hc0_h, hc1_h, hc2_h, hc3_h)
    os_ = (oc0_h, oc1_h, oc2_h, oc3_h)
    for cc in range(NC):
        @pl.when(c == cc)
        def _(cc=cc):
            for fp in range(2):
                f = cc * 2 + fp
                _fpass128(src2_h, dst2_h, ex_h, hs[f], os_[f], out_sh, s, f,
                          z_h, srcv2, dstv2, exv6, rowsA, gsA, ssA,
                          rowsB, gsB, wbv)


def _msg1(src2, dst2, ex1, hc0, hc1, hc2, hc3, z128):
    f32 = jnp.float32
    return pl.kernel(
        _msg1_body,
        out_type=[jax.ShapeDtypeStruct((NP, 128), f32)] * 4,
        mesh=_mesh(),
        compiler_params=pltpu.CompilerParams(use_tc_tiling_on_sc=False),
        scratch_types=[
            pltpu.VMEM_SHARED((NP, 128), f32),
            pltpu.VMEM((SG, CHM), jnp.int32),
            pltpu.VMEM((SG, CHM), jnp.int32),
            pltpu.VMEM((SG * CHM, 16), f32),
            pltpu.VMEM((CHM, 128), f32),
            pltpu.VMEM((CHM, 128), f32),
            pltpu.SemaphoreType.DMA,
            pltpu.SemaphoreType.DMA,
            pltpu.SemaphoreType.DMA,
            pltpu.VMEM((16, 128), f32),
        ],
    )(src2, dst2, ex1, hc0, hc1, hc2, hc3, z128)


def _l2_body(src_h, dst_h, as_h, ad_h, g_h, h2_h, z16_h, z64_h,
             op_h, dp_h,
             out_sh, den_sh, srcv, dstv, g1v, g2v, exv, rows_v, gv, wbv):
    c = lax.axis_index("c")
    s = lax.axis_index("s")
    w = s * NC + c
    pltpu.sync_copy(z16_h, den_sh.at[pl.ds(s * STRIPE, STRIPE)])
    pltpu.sync_copy(z64_h, out_sh.at[pl.ds(s * STRIPE, STRIPE)])
    pltpu.sync_copy(g_h, gv)
    plsc.subcore_barrier()
    gvec = gv[...]
    base = w * (EP // 32)

    @pl.loop(0, (EP // 32) // CH)
    def _chunk(t):
        eb = base + t * CH
        pltpu.sync_copy(src_h.at[pl.ds(eb, CH)], srcv)
        pltpu.sync_copy(dst_h.at[pl.ds(eb, CH)], dstv)
        pltpu.sync_copy(as_h.at[srcv], g1v)
        pltpu.sync_copy(ad_h.at[dstv], g2v)

        @pl.loop(0, CH)
        def _edge(j):
            a = g1v[j] + g2v[j]
            a = jnp.where(a >= 0.0, a, 0.2 * a)
            exv[j] = jnp.exp(a - gvec)

        pltpu.sync_copy(exv, den_sh.at[dstv], add=True)
        pltpu.sync_copy(h2_h.at[srcv], rows_v)

        @pl.loop(0, CH)
        def _scale(j):
            sc = exv[j, :][0]
            for k in range(HID // 16):
                rows_v[j, pl.ds(k * 16, 16)] = rows_v[j, pl.ds(k * 16, 16)] * sc

        pltpu.sync_copy(rows_v, out_sh.at[dstv], add=True)

    plsc.subcore_barrier()
    pltpu.sync_copy(den_sh.at[pl.ds(s * STRIPE, STRIPE)], wbv)
    pltpu.sync_copy(wbv, dp_h.at[pl.ds(c * NP + s * STRIPE, STRIPE)])
    rb = s * STRIPE
    for off, sz in ((0, CH), (CH, CH), (2 * CH, CH), (3 * CH, CH),
                    (4 * CH, STRIPE - 4 * CH)):
        pltpu.sync_copy(out_sh.at[pl.ds(rb + off, sz)],
                        rows_v.at[pl.ds(0, sz)])
        pltpu.sync_copy(rows_v.at[pl.ds(0, sz)],
                        op_h.at[pl.ds(c * NP + rb + off, sz)])


def _l2(src, dst, as2, ad2, g16, h2, z16, z64):
    f32 = jnp.float32
    return pl.kernel(
        _l2_body,
        out_type=[jax.ShapeDtypeStruct((2 * NP, HID), f32),
                  jax.ShapeDtypeStruct((2 * NP, 16), f32)],
        mesh=_mesh(),
        compiler_params=pltpu.CompilerParams(use_tc_tiling_on_sc=False),
        scratch_types=[
            pltpu.VMEM_SHARED((NP, HID), f32),
            pltpu.VMEM_SHARED((NP, 16), f32),
            pltpu.VMEM((CH,), jnp.int32),
            pltpu.VMEM((CH,), jnp.int32),
            pltpu.VMEM((CH, 16), f32),
            pltpu.VMEM((CH, 16), f32),
            pltpu.VMEM((CH, 16), f32),
            pltpu.VMEM((CH, HID), f32),
            pltpu.VMEM((16,), f32),
            pltpu.VMEM((STRIPE, 16), f32),
        ],
    )(src, dst, as2, ad2, g16, h2, z16, z64)


def _blockdiag16(att):
    h, d = att.shape
    eye = jnp.eye(8, dtype=att.dtype)[:h]
    m = att[:, :, None] * eye[:, None, :]
    m = m.reshape(h * d, 8)
    if h * d < 512:
        m = jnp.pad(m, ((0, 512 - h * d), (0, 0)))
    return jnp.concatenate([m, m], axis=1)


def kernel(x, edge_index, W1, att_src1, att_dst1, b1, ln1_w, ln1_b,
           W2, att_src2, att_dst2, b2, ln2_w, ln2_b):
    f32 = jnp.float32
    src = edge_index[0].astype(jnp.int32)
    dst = edge_index[1].astype(jnp.int32)
    loop = jnp.arange(N, dtype=jnp.int32)
    pad = jnp.full((EP - ET,), SENT, jnp.int32)
    src = jnp.concatenate([src, loop, pad])
    dst = jnp.concatenate([dst, loop, pad])

    xp = jnp.pad(x, ((0, NP - N), (0, 0)))

    ams1 = _blockdiag16(att_src1.reshape(HEADS, HID))
    amd1 = _blockdiag16(att_dst1.reshape(HEADS, HID))
    hc0, hc1, hc2, hc3, as1, ad1, gs1, gd1 = _tc1(xp, W1, ams1, amd1)
    g1 = jnp.max(gs1, axis=0) + jnp.max(gd1, axis=0)

    z16 = jnp.zeros((STRIPE, 16), f32)
    z64 = jnp.zeros((STRIPE, HID), f32)
    z128 = jnp.zeros((STRIPE, 128), f32)

    ex1, den1 = _att1(src, dst, as1, ad1, g1, z16)
    src2 = src.reshape(EP // CHM, CHM)
    dst2 = dst.reshape(EP // CHM, CHM)
    oc0, oc1, oc2, oc3 = _msg1(src2, dst2, ex1, hc0, hc1, hc2, hc3, z128)

    ex16 = jnp.concatenate(
        [jnp.repeat(jnp.eye(8, dtype=f32), HID, axis=1),
         jnp.zeros((8, 512), f32)], axis=0)
    a2s = jnp.tile(att_src2.reshape(HID, 1), (1, 16))
    a2d = jnp.tile(att_dst2.reshape(HID, 1), (1, 16))
    h2, as2, ad2, gs2, gd2 = _tc2(
        oc0, oc1, oc2, oc3, den1[:NP], den1[NP:], b1.reshape(1, 512),
        ln1_w.reshape(1, 512), ln1_b.reshape(1, 512), ex16, W2, a2s, a2d)
    g2 = jnp.max(gs2, axis=0) + jnp.max(gd2, axis=0)

    op2, dp2 = _l2(src, dst, as2, ad2, g2, h2, z16, z64)

    sel = jnp.zeros((16, HID), f32).at[0, :].set(1.0)
    out = _tc3(op2[:NP], op2[NP:], dp2[:NP], dp2[NP:], sel,
               b2.reshape(1, HID), ln2_w.reshape(1, HID),
               ln2_b.reshape(1, HID))
    return out[:N]


# speedup vs baseline: 1.4021x; 1.0516x over previous
import jax
import jax.numpy as jnp
from jax import lax
from jax.experimental import pallas as pl
from jax.experimental.pallas import tpu as pltpu
from jax.experimental.pallas import tpu_sc as plsc

N = 10000
E = 320000
IN_DIM = 128
HID = 64
HEADS = 8
EPS = 1e-5

NP = 10240
ET = E + N
CH = 128
EP = 331776
SENT = NP - 1
NEG = -1e30

NC = 2
NS = 16
HALF = NP // 2
OB = HALF + 128
STRIPE = NP // NS

BR = 2560
GR = NP // BR

def _mesh():
    return plsc.VectorSubcoreMesh(core_axis_name="c", subcore_axis_name="s",
                                  num_cores=NC, num_subcores=NS)


def _tc1_body(x_ref, w1_ref, ams_ref, amd_ref,
              hc0_ref, hc1_ref, hc2_ref, hc3_ref,
              as_ref, ad_ref, gs_ref, gd_ref):
    i = pl.program_id(0)
    h = jnp.dot(x_ref[...], w1_ref[...], preferred_element_type=jnp.float32)
    hc0_ref[...] = h[:, 0:128]
    hc1_ref[...] = h[:, 128:256]
    hc2_ref[...] = h[:, 256:384]
    hc3_ref[...] = h[:, 384:512]
    asv = jnp.dot(h, ams_ref[...], preferred_element_type=jnp.float32)
    adv = jnp.dot(h, amd_ref[...], preferred_element_type=jnp.float32)
    row = i * BR + lax.broadcasted_iota(jnp.int32, (BR, 16), 0)
    asv = jnp.where(row < N, asv, NEG)
    adv = jnp.where(row < N, adv, NEG)
    as_ref[...] = asv
    ad_ref[...] = adv
    bs = jnp.max(asv, axis=0, keepdims=True)
    bd = jnp.max(adv, axis=0, keepdims=True)

    @pl.when(i == 0)
    def _():
        gs_ref[...] = jnp.broadcast_to(bs, (8, 16))
        gd_ref[...] = jnp.broadcast_to(bd, (8, 16))

    @pl.when(i > 0)
    def _():
        gs_ref[...] = jnp.maximum(gs_ref[...], bs)
        gd_ref[...] = jnp.maximum(gd_ref[...], bd)


def _tc1(xp, W1, ams, amd):
    f32 = jnp.float32
    blk = lambda shp: pl.BlockSpec(shp, lambda i: (i, 0))
    whole = lambda shp: pl.BlockSpec(shp, lambda i: (0, 0))
    return pl.pallas_call(
        _tc1_body,
        grid=(GR,),
        in_specs=[blk((BR, IN_DIM)), whole((IN_DIM, 512)),
                  whole((512, 16)), whole((512, 16))],
        out_specs=[blk((BR, 128)), blk((BR, 128)), blk((BR, 128)),
                   blk((BR, 128)), blk((BR, 16)), blk((BR, 16)),
                   whole((8, 16)), whole((8, 16))],
        out_shape=[jax.ShapeDtypeStruct((NP, 128), f32),
                   jax.ShapeDtypeStruct((NP, 128), f32),
                   jax.ShapeDtypeStruct((NP, 128), f32),
                   jax.ShapeDtypeStruct((NP, 128), f32),
                   jax.ShapeDtypeStruct((NP, 16), f32),
                   jax.ShapeDtypeStruct((NP, 16), f32),
                   jax.ShapeDtypeStruct((8, 16), f32),
                   jax.ShapeDtypeStruct((8, 16), f32)],
    )(xp, W1, ams, amd)


def _ln_elu(o, lw, lb):
    mu = jnp.mean(o, axis=1, keepdims=True)
    var = jnp.mean((o - mu) ** 2, axis=1, keepdims=True)
    o = (o - mu) / jnp.sqrt(var + EPS) * lw + lb
    return jnp.where(o > 0, o, jnp.exp(jnp.minimum(o, 0.0)) - 1.0)


def _tc2_body(oa_ref, ob_ref, oc_ref, od_ref, d0_ref, d1_ref, b1_ref, lw_ref, lb_ref,
              ex16_ref, w2_ref, a2s_ref, a2d_ref,
              h2_ref, as2_ref, ad2_ref, gs_ref, gd_ref):
    i = pl.program_id(0)
    den = d0_ref[...] + d1_ref[...]
    dinv = 1.0 / (den + 1e-16)
    dinv512 = jnp.dot(dinv, ex16_ref[...], preferred_element_type=jnp.float32)
    o = jnp.concatenate([oa_ref[...], ob_ref[...], oc_ref[...], od_ref[...]],
                        axis=1)
    o = o * dinv512 + b1_ref[...]
    o = _ln_elu(o, lw_ref[...], lb_ref[...])
    h2 = jnp.dot(o, w2_ref[...], preferred_element_type=jnp.float32)
    h2_ref[...] = h2
    asv = jnp.dot(h2, a2s_ref[...], preferred_element_type=jnp.float32)
    adv = jnp.dot(h2, a2d_ref[...], preferred_element_type=jnp.float32)
    row = i * BR + lax.broadcasted_iota(jnp.int32, (BR, 16), 0)
    asv = jnp.where(row < N, asv, NEG)
    adv = jnp.where(row < N, adv, NEG)
    as2_ref[...] = asv
    ad2_ref[...] = adv
    bs = jnp.max(asv, axis=0, keepdims=True)
    bd = jnp.max(adv, axis=0, keepdims=True)

    @pl.when(i == 0)
    def _():
        gs_ref[...] = jnp.broadcast_to(bs, (8, 16))
        gd_ref[...] = jnp.broadcast_to(bd, (8, 16))

    @pl.when(i > 0)
    def _():
        gs_ref[...] = jnp.maximum(gs_ref[...], bs)
        gd_ref[...] = jnp.maximum(gd_ref[...], bd)


def _tc2(oa, ob, oc, od, d0, d1, b1r, lw, lb, ex16, W2, a2s, a2d):
    f32 = jnp.float32
    blk = lambda shp: pl.BlockSpec(shp, lambda i: (i, 0))
    whole = lambda shp: pl.BlockSpec(shp, lambda i: (0, 0))
    return pl.pallas_call(
        _tc2_body,
        grid=(GR,),
        in_specs=[blk((BR, 128)), blk((BR, 128)), blk((BR, 128)), blk((BR, 128)),
                  blk((BR, 16)), blk((BR, 16)),
                  whole((1, 512)), whole((1, 512)), whole((1, 512)),
                  whole((16, 512)), whole((512, HID)),
                  whole((HID, 16)), whole((HID, 16))],
        out_specs=[blk((BR, HID)), blk((BR, 16)), blk((BR, 16)),
                   whole((8, 16)), whole((8, 16))],
        out_shape=[jax.ShapeDtypeStruct((NP, HID), f32),
                   jax.ShapeDtypeStruct((NP, 16), f32),
                   jax.ShapeDtypeStruct((NP, 16), f32),
                   jax.ShapeDtypeStruct((8, 16), f32),
                   jax.ShapeDtypeStruct((8, 16), f32)],
    )(oa, ob, oc, od, d0, d1, b1r, lw, lb, ex16, W2, a2s, a2d)


def _tc3_body(p0_ref, p1_ref, d0_ref, d1_ref, sel_ref, b2_ref, lw_ref, lb_ref,
              out_ref):
    den = d0_ref[...] + d1_ref[...]
    dinv = 1.0 / (den + 1e-16)
    dinv64 = jnp.dot(dinv, sel_ref[...], preferred_element_type=jnp.float32)
    o = (p0_ref[...] + p1_ref[...]) * dinv64 + b2_ref[...]
    out_ref[...] = _ln_elu(o, lw_ref[...], lb_ref[...])


def _tc3(p0, p1, d0, d1, sel, b2r, lw, lb):
    blk = lambda shp: pl.BlockSpec(shp, lambda i: (i, 0))
    whole = lambda shp: pl.BlockSpec(shp, lambda i: (0, 0))
    return pl.pallas_call(
        _tc3_body,
        grid=(GR,),
        in_specs=[blk((BR, HID)), blk((BR, HID)), blk((BR, 16)), blk((BR, 16)),
                  whole((16, HID)), whole((1, HID)), whole((1, HID)),
                  whole((1, HID))],
        out_specs=blk((BR, HID)),
        out_shape=jax.ShapeDtypeStruct((NP, HID), jnp.float32),
    )(p0, p1, d0, d1, sel, b2r, lw, lb)


def _att1_body(src_h, dst_h, as_h, ad_h, g_h, z_h,
               ex_h, den_h,
               den_sh, srcv, dstv, g1v, g2v, exv, gv, wbv):
    c = lax.axis_index("c")
    s = lax.axis_index("s")
    w = s * NC + c
    pltpu.sync_copy(z_h, den_sh.at[pl.ds(s * STRIPE, STRIPE)])
    pltpu.sync_copy(g_h, gv)
    plsc.subcore_barrier()
    gvec = gv[...]
    base = w * (EP // 32)

    @pl.loop(0, (EP // 32) // CH)
    def _chunk(t):
        eb = base + t * CH
        pltpu.sync_copy(src_h.at[pl.ds(eb, CH)], srcv)
        pltpu.sync_copy(dst_h.at[pl.ds(eb, CH)], dstv)
        pltpu.sync_copy(as_h.at[srcv], g1v)
        pltpu.sync_copy(ad_h.at[dstv], g2v)

        @pl.loop(0, CH)
        def _edge(j):
            a = g1v[j] + g2v[j]
            a = jnp.where(a >= 0.0, a, 0.2 * a)
            exv[j] = jnp.exp(a - gvec)

        pltpu.sync_copy(exv, ex_h.at[pl.ds(eb, CH)])
        pltpu.sync_copy(exv, den_sh.at[dstv], add=True)

    plsc.subcore_barrier()
    pltpu.sync_copy(den_sh.at[pl.ds(s * STRIPE, STRIPE)], wbv)
    pltpu.sync_copy(wbv, den_h.at[pl.ds(c * NP + s * STRIPE, STRIPE)])


def _att1(src, dst, as16, ad16, g16, z16):
    f32 = jnp.float32
    return pl.kernel(
        _att1_body,
        out_type=[jax.ShapeDtypeStruct((EP, 16), f32),
                  jax.ShapeDtypeStruct((2 * NP, 16), f32)],
        mesh=_mesh(),
        compiler_params=pltpu.CompilerParams(use_tc_tiling_on_sc=False),
        scratch_types=[
            pltpu.VMEM_SHARED((NP, 16), f32),
            pltpu.VMEM((CH,), jnp.int32),
            pltpu.VMEM((CH,), jnp.int32),
            pltpu.VMEM((CH, 16), f32),
            pltpu.VMEM((CH, 16), f32),
            pltpu.VMEM((CH, 16), f32),
            pltpu.VMEM((16,), f32),
            pltpu.VMEM((STRIPE, 16), f32),
        ],
    )(src, dst, as16, ad16, g16, z16)


CHM = 128
SG = 6


def _fpass128(src2_h, dst2_h, ex_h, h_h, o_h, out_sh, s, f, z_h,
              srcv2, dstv2, exv6, rowsA, gsA, ssA, rowsB, gsB, wbv):
    NCHT = (EP // 16) // CHM
    pltpu.sync_copy(z_h, out_sh.at[pl.ds(s * STRIPE, STRIPE)])
    plsc.subcore_barrier()
    ebase = s * (EP // 16)
    crow = s * NCHT

    def scale(t, rowsp):
        @pl.loop(0, CHM, unroll=2)
        def _edge(j):
            exrow = exv6[t * CHM + j, :]
            for k in range(8):
                sc = exrow[f * 2 + (k // 4)]
                rowsp[j, pl.ds(k * 16, 16)] = rowsp[j, pl.ds(k * 16, 16)] * sc

    @pl.loop(0, NCHT // SG)
    def _u(u):
        pltpu.sync_copy(src2_h.at[pl.ds(crow + u * SG, SG)], srcv2)
        pltpu.sync_copy(dst2_h.at[pl.ds(crow + u * SG, SG)], dstv2)
        pltpu.sync_copy(ex_h.at[pl.ds(ebase + u * SG * CHM, SG * CHM)], exv6)
        pltpu.async_copy(h_h.at[srcv2.at[0]], rowsA, gsA)
        for p in range(SG // 2):
            tA, tB = 2 * p, 2 * p + 1
            pltpu.async_copy(h_h.at[srcv2.at[tB]], rowsB, gsB)
            pltpu.make_async_copy(h_h.at[srcv2.at[tA]], rowsA, gsA).wait()
            scale(tA, rowsA)
            pltpu.async_copy(rowsA, out_sh.at[dstv2.at[tA]], ssA, add=True)
            if p < SG // 2 - 1:
                pltpu.make_async_copy(rowsA, out_sh.at[dstv2.at[tA]],
                                      ssA).wait()
                pltpu.async_copy(h_h.at[srcv2.at[tA + 2]], rowsA, gsA)
            pltpu.make_async_copy(h_h.at[srcv2.at[tB]], rowsB, gsB).wait()
            scale(tB, rowsB)
            pltpu.sync_copy(rowsB, out_sh.at[dstv2.at[tB]], add=True)
        pltpu.make_async_copy(rowsA, out_sh.at[dstv2.at[SG - 2]], ssA).wait()

    plsc.subcore_barrier()
    rb = s * STRIPE
    for q in range(STRIPE // 16):
        pltpu.sync_copy(out_sh.at[pl.ds(rb + q * 16, 16)], wbv)
        pltpu.sync_copy(wbv, o_h.at[pl.ds(rb + q * 16, 16)])
    plsc.subcore_barrier()


def _msg1_body(src2_h, dst2_h, ex_h, hc0_h, hc1_h, hc2_h, hc3_h, z_h,
               oc0_h, oc1_h, oc2_h, oc3_h,
               out_sh, srcv2, dstv2, exv6, rowsA, rowsB,
               gsA, ssA, gsB, wbv):
    c = lax.axis_index("c")
    s = lax.axis_index("s")
    hs = (hc0_h, hc1_h, hc2_h, hc3_h)
    os_ = (oc0_h, oc1_h, oc2_h, oc3_h)
    for cc in range(NC):
        @pl.when(c == cc)
        def _(cc=cc):
            for fp in range(2):
                f = cc * 2 + fp
                _fpass128(src2_h, dst2_h, ex_h, hs[f], os_[f], out_sh, s, f,
                          z_h, srcv2, dstv2, exv6, rowsA, gsA, ssA,
                          rowsB, gsB, wbv)


def _msg1(src2, dst2, ex1, hc0, hc1, hc2, hc3, z128):
    f32 = jnp.float32
    return pl.kernel(
        _msg1_body,
        out_type=[jax.ShapeDtypeStruct((NP, 128), f32)] * 4,
        mesh=_mesh(),
        compiler_params=pltpu.CompilerParams(use_tc_tiling_on_sc=False),
        scratch_types=[
            pltpu.VMEM_SHARED((NP, 128), f32),
            pltpu.VMEM((SG, CHM), jnp.int32),
            pltpu.VMEM((SG, CHM), jnp.int32),
            pltpu.VMEM((SG * CHM, 16), f32),
            pltpu.VMEM((CHM, 128), f32),
            pltpu.VMEM((CHM, 128), f32),
            pltpu.SemaphoreType.DMA,
            pltpu.SemaphoreType.DMA,
            pltpu.SemaphoreType.DMA,
            pltpu.VMEM((16, 128), f32),
        ],
    )(src2, dst2, ex1, hc0, hc1, hc2, hc3, z128)


CHL = 96
SGL = 4


def _l2_body(src3_h, dst3_h, as_h, ad_h, g_h, h2_h, z16_h, z64_h,
             op_h, dp_h,
             out_sh, den_sh, srcv2, dstv2,
             g1A, g2A, exA, rowsA, g1B, g2B, exB, rowsB,
             sA1, sA2, sAr, sB1, sB2, sBr, gv, wbv, wb64):
    c = lax.axis_index("c")
    s = lax.axis_index("s")
    w = s * NC + c
    pltpu.sync_copy(z16_h, den_sh.at[pl.ds(s * STRIPE, STRIPE)])
    pltpu.sync_copy(z64_h, out_sh.at[pl.ds(s * STRIPE, STRIPE)])
    pltpu.sync_copy(g_h, gv)
    plsc.subcore_barrier()
    gvec = gv[...]
    NCHT = (EP // 32) // CHL
    crow = w * NCHT

    def fire3(t, g1p, g2p, rowsp, s1, s2, sr):
        pltpu.async_copy(as_h.at[srcv2.at[t]], g1p, s1)
        pltpu.async_copy(ad_h.at[dstv2.at[t]], g2p, s2)
        pltpu.async_copy(h2_h.at[srcv2.at[t]], rowsp, sr)

    def wait3(t, g1p, g2p, rowsp, s1, s2, sr):
        pltpu.make_async_copy(as_h.at[srcv2.at[t]], g1p, s1).wait()
        pltpu.make_async_copy(ad_h.at[dstv2.at[t]], g2p, s2).wait()
        pltpu.make_async_copy(h2_h.at[srcv2.at[t]], rowsp, sr).wait()

    def work(t, g1p, g2p, exp_, rowsp):
        @pl.loop(0, CHL, unroll=2)
        def _edge(j):
            a = g1p[j, :] + g2p[j, :]
            a = jnp.where(a >= 0.0, a, 0.2 * a)
            e = jnp.exp(a - gvec)
            exp_[j, :] = e
            sc = e[0]
            for k in range(HID // 16):
                rowsp[j, pl.ds(k * 16, 16)] = rowsp[j, pl.ds(k * 16, 16)] * sc

        pltpu.sync_copy(exp_, den_sh.at[dstv2.at[t]], add=True)
        pltpu.sync_copy(rowsp, out_sh.at[dstv2.at[t]], add=True)

    @pl.loop(0, NCHT // SGL)
    def _u(u):
        pltpu.sync_copy(src3_h.at[pl.ds(crow + u * SGL, SGL)], srcv2)
        pltpu.sync_copy(dst3_h.at[pl.ds(crow + u * SGL, SGL)], dstv2)
        fire3(0, g1A, g2A, rowsA, sA1, sA2, sAr)
        for pp in range(SGL // 2):
            tA, tB = 2 * pp, 2 * pp + 1
            fire3(tB, g1B, g2B, rowsB, sB1, sB2, sBr)
            wait3(tA, g1A, g2A, rowsA, sA1, sA2, sAr)
            work(tA, g1A, g2A, exA, rowsA)
            if pp < SGL // 2 - 1:
                fire3(tA + 2, g1A, g2A, rowsA, sA1, sA2, sAr)
            wait3(tB, g1B, g2B, rowsB, sB1, sB2, sBr)
            work(tB, g1B, g2B, exB, rowsB)

    plsc.subcore_barrier()
    pltpu.sync_copy(den_sh.at[pl.ds(s * STRIPE, STRIPE)], wbv)
    pltpu.sync_copy(wbv, dp_h.at[pl.ds(c * NP + s * STRIPE, STRIPE)])
    rb = s * STRIPE
    for q in range(STRIPE // 16):
        pltpu.sync_copy(out_sh.at[pl.ds(rb + q * 16, 16)], wb64)
        pltpu.sync_copy(wb64, op_h.at[pl.ds(c * NP + rb + q * 16, 16)])


def _l2(src3, dst3, as2, ad2, g16, h2, z16, z64):
    f32 = jnp.float32
    return pl.kernel(
        _l2_body,
        out_type=[jax.ShapeDtypeStruct((2 * NP, HID), f32),
                  jax.ShapeDtypeStruct((2 * NP, 16), f32)],
        mesh=_mesh(),
        compiler_params=pltpu.CompilerParams(use_tc_tiling_on_sc=False),
        scratch_types=[
            pltpu.VMEM_SHARED((NP, HID), f32),
            pltpu.VMEM_SHARED((NP, 16), f32),
            pltpu.VMEM((SGL, CHL), jnp.int32),
            pltpu.VMEM((SGL, CHL), jnp.int32),
            pltpu.VMEM((CHL, 16), f32),
            pltpu.VMEM((CHL, 16), f32),
            pltpu.VMEM((CHL, 16), f32),
            pltpu.VMEM((CHL, HID), f32),
            pltpu.VMEM((CHL, 16), f32),
            pltpu.VMEM((CHL, 16), f32),
            pltpu.VMEM((CHL, 16), f32),
            pltpu.VMEM((CHL, HID), f32),
            pltpu.SemaphoreType.DMA,
            pltpu.SemaphoreType.DMA,
            pltpu.SemaphoreType.DMA,
            pltpu.SemaphoreType.DMA,
            pltpu.SemaphoreType.DMA,
            pltpu.SemaphoreType.DMA,
            pltpu.VMEM((16,), f32),
            pltpu.VMEM((STRIPE, 16), f32),
            pltpu.VMEM((16, HID), f32),
        ],
    )(src3, dst3, as2, ad2, g16, h2, z16, z64)


def _blockdiag16(att):
    h, d = att.shape
    eye = jnp.eye(8, dtype=att.dtype)[:h]
    m = att[:, :, None] * eye[:, None, :]
    m = m.reshape(h * d, 8)
    if h * d < 512:
        m = jnp.pad(m, ((0, 512 - h * d), (0, 0)))
    return jnp.concatenate([m, m], axis=1)


def kernel(x, edge_index, W1, att_src1, att_dst1, b1, ln1_w, ln1_b,
           W2, att_src2, att_dst2, b2, ln2_w, ln2_b):
    f32 = jnp.float32
    src = edge_index[0].astype(jnp.int32)
    dst = edge_index[1].astype(jnp.int32)
    loop = jnp.arange(N, dtype=jnp.int32)
    pad = jnp.full((EP - ET,), SENT, jnp.int32)
    src = jnp.concatenate([src, loop, pad])
    dst = jnp.concatenate([dst, loop, pad])

    xp = jnp.pad(x, ((0, NP - N), (0, 0)))

    ams1 = _blockdiag16(att_src1.reshape(HEADS, HID))
    amd1 = _blockdiag16(att_dst1.reshape(HEADS, HID))
    hc0, hc1, hc2, hc3, as1, ad1, gs1, gd1 = _tc1(xp, W1, ams1, amd1)
    g1 = jnp.max(gs1, axis=0) + jnp.max(gd1, axis=0)

    z16 = jnp.zeros((STRIPE, 16), f32)
    z64 = jnp.zeros((STRIPE, HID), f32)
    z128 = jnp.zeros((STRIPE, 128), f32)

    ex1, den1 = _att1(src, dst, as1, ad1, g1, z16)
    src2 = src.reshape(EP // CHM, CHM)
    dst2 = dst.reshape(EP // CHM, CHM)
    oc0, oc1, oc2, oc3 = _msg1(src2, dst2, ex1, hc0, hc1, hc2, hc3, z128)

    ex16 = jnp.concatenate(
        [jnp.repeat(jnp.eye(8, dtype=f32), HID, axis=1),
         jnp.zeros((8, 512), f32)], axis=0)
    a2s = jnp.tile(att_src2.reshape(HID, 1), (1, 16))
    a2d = jnp.tile(att_dst2.reshape(HID, 1), (1, 16))
    h2, as2, ad2, gs2, gd2 = _tc2(
        oc0, oc1, oc2, oc3, den1[:NP], den1[NP:], b1.reshape(1, 512),
        ln1_w.reshape(1, 512), ln1_b.reshape(1, 512), ex16, W2, a2s, a2d)
    g2 = jnp.max(gs2, axis=0) + jnp.max(gd2, axis=0)

    src3 = src.reshape(EP // CHL, CHL)
    dst3 = dst.reshape(EP // CHL, CHL)
    op2, dp2 = _l2(src3, dst3, as2, ad2, g2, h2, z16, z64)

    sel = jnp.zeros((16, HID), f32).at[0, :].set(1.0)
    out = _tc3(op2[:NP], op2[NP:], dp2[:NP], dp2[NP:], sel,
               b2.reshape(1, HID), ln2_w.reshape(1, HID),
               ln2_b.reshape(1, HID))
    return out[:N]


# speedup vs baseline: 1.4184x; 1.0116x over previous
import jax
import jax.numpy as jnp
from jax import lax
from jax.experimental import pallas as pl
from jax.experimental.pallas import tpu as pltpu
from jax.experimental.pallas import tpu_sc as plsc

N = 10000
E = 320000
IN_DIM = 128
HID = 64
HEADS = 8
EPS = 1e-5

NP = 10240
ET = E + N
CH = 128
EP = 331776
SENT = NP - 1
NEG = -1e30

NC = 2
NS = 16
HALF = NP // 2
OB = HALF + 128
STRIPE = NP // NS

BR = 2560
GR = NP // BR

def _mesh():
    return plsc.VectorSubcoreMesh(core_axis_name="c", subcore_axis_name="s",
                                  num_cores=NC, num_subcores=NS)


def _tc1_body(x_ref, w1_ref, ams_ref, amd_ref,
              hc0_ref, hc1_ref, hc2_ref, hc3_ref,
              as_ref, ad_ref, gs_ref, gd_ref):
    i = pl.program_id(0)
    h = jnp.dot(x_ref[...], w1_ref[...], preferred_element_type=jnp.float32)
    hc0_ref[...] = h[:, 0:128]
    hc1_ref[...] = h[:, 128:256]
    hc2_ref[...] = h[:, 256:384]
    hc3_ref[...] = h[:, 384:512]
    asv = jnp.dot(h, ams_ref[...], preferred_element_type=jnp.float32)
    adv = jnp.dot(h, amd_ref[...], preferred_element_type=jnp.float32)
    row = i * BR + lax.broadcasted_iota(jnp.int32, (BR, 16), 0)
    asv = jnp.where(row < N, asv, NEG)
    adv = jnp.where(row < N, adv, NEG)
    as_ref[...] = asv
    ad_ref[...] = adv
    bs = jnp.max(asv, axis=0, keepdims=True)
    bd = jnp.max(adv, axis=0, keepdims=True)

    @pl.when(i == 0)
    def _():
        gs_ref[...] = jnp.broadcast_to(bs, (8, 16))
        gd_ref[...] = jnp.broadcast_to(bd, (8, 16))

    @pl.when(i > 0)
    def _():
        gs_ref[...] = jnp.maximum(gs_ref[...], bs)
        gd_ref[...] = jnp.maximum(gd_ref[...], bd)


def _tc1(xp, W1, ams, amd):
    f32 = jnp.float32
    blk = lambda shp: pl.BlockSpec(shp, lambda i: (i, 0))
    whole = lambda shp: pl.BlockSpec(shp, lambda i: (0, 0))
    return pl.pallas_call(
        _tc1_body,
        grid=(GR,),
        in_specs=[blk((BR, IN_DIM)), whole((IN_DIM, 512)),
                  whole((512, 16)), whole((512, 16))],
        out_specs=[blk((BR, 128)), blk((BR, 128)), blk((BR, 128)),
                   blk((BR, 128)), blk((BR, 16)), blk((BR, 16)),
                   whole((8, 16)), whole((8, 16))],
        out_shape=[jax.ShapeDtypeStruct((NP, 128), f32),
                   jax.ShapeDtypeStruct((NP, 128), f32),
                   jax.ShapeDtypeStruct((NP, 128), f32),
                   jax.ShapeDtypeStruct((NP, 128), f32),
                   jax.ShapeDtypeStruct((NP, 16), f32),
                   jax.ShapeDtypeStruct((NP, 16), f32),
                   jax.ShapeDtypeStruct((8, 16), f32),
                   jax.ShapeDtypeStruct((8, 16), f32)],
    )(xp, W1, ams, amd)


def _ln_elu(o, lw, lb):
    mu = jnp.mean(o, axis=1, keepdims=True)
    var = jnp.mean((o - mu) ** 2, axis=1, keepdims=True)
    o = (o - mu) / jnp.sqrt(var + EPS) * lw + lb
    return jnp.where(o > 0, o, jnp.exp(jnp.minimum(o, 0.0)) - 1.0)


def _tc2_body(oa_ref, ob_ref, oc_ref, od_ref, d0_ref, d1_ref, b1_ref, lw_ref, lb_ref,
              ex16_ref, w2_ref, a2s_ref, a2d_ref,
              h2_ref, as2_ref, ad2_ref, gs_ref, gd_ref):
    i = pl.program_id(0)
    den = d0_ref[...] + d1_ref[...]
    dinv = 1.0 / (den + 1e-16)
    dinv512 = jnp.dot(dinv, ex16_ref[...], preferred_element_type=jnp.float32)
    o = jnp.concatenate([oa_ref[...], ob_ref[...], oc_ref[...], od_ref[...]],
                        axis=1)
    o = o * dinv512 + b1_ref[...]
    o = _ln_elu(o, lw_ref[...], lb_ref[...])
    h2 = jnp.dot(o, w2_ref[...], preferred_element_type=jnp.float32)
    h2_ref[...] = h2
    asv = jnp.dot(h2, a2s_ref[...], preferred_element_type=jnp.float32)
    adv = jnp.dot(h2, a2d_ref[...], preferred_element_type=jnp.float32)
    row = i * BR + lax.broadcasted_iota(jnp.int32, (BR, 16), 0)
    asv = jnp.where(row < N, asv, NEG)
    adv = jnp.where(row < N, adv, NEG)
    as2_ref[...] = asv
    ad2_ref[...] = adv
    bs = jnp.max(asv, axis=0, keepdims=True)
    bd = jnp.max(adv, axis=0, keepdims=True)

    @pl.when(i == 0)
    def _():
        gs_ref[...] = jnp.broadcast_to(bs, (8, 16))
        gd_ref[...] = jnp.broadcast_to(bd, (8, 16))

    @pl.when(i > 0)
    def _():
        gs_ref[...] = jnp.maximum(gs_ref[...], bs)
        gd_ref[...] = jnp.maximum(gd_ref[...], bd)


def _tc2(oa, ob, oc, od, d0, d1, b1r, lw, lb, ex16, W2, a2s, a2d):
    f32 = jnp.float32
    blk = lambda shp: pl.BlockSpec(shp, lambda i: (i, 0))
    whole = lambda shp: pl.BlockSpec(shp, lambda i: (0, 0))
    return pl.pallas_call(
        _tc2_body,
        grid=(GR,),
        in_specs=[blk((BR, 128)), blk((BR, 128)), blk((BR, 128)), blk((BR, 128)),
                  blk((BR, 16)), blk((BR, 16)),
                  whole((1, 512)), whole((1, 512)), whole((1, 512)),
                  whole((16, 512)), whole((512, HID)),
                  whole((HID, 16)), whole((HID, 16))],
        out_specs=[blk((BR, HID)), blk((BR, 16)), blk((BR, 16)),
                   whole((8, 16)), whole((8, 16))],
        out_shape=[jax.ShapeDtypeStruct((NP, HID), f32),
                   jax.ShapeDtypeStruct((NP, 16), f32),
                   jax.ShapeDtypeStruct((NP, 16), f32),
                   jax.ShapeDtypeStruct((8, 16), f32),
                   jax.ShapeDtypeStruct((8, 16), f32)],
    )(oa, ob, oc, od, d0, d1, b1r, lw, lb, ex16, W2, a2s, a2d)


def _tc3_body(p0_ref, p1_ref, d0_ref, d1_ref, sel_ref, b2_ref, lw_ref, lb_ref,
              out_ref):
    den = d0_ref[...] + d1_ref[...]
    dinv = 1.0 / (den + 1e-16)
    dinv64 = jnp.dot(dinv, sel_ref[...], preferred_element_type=jnp.float32)
    o = (p0_ref[...] + p1_ref[...]) * dinv64 + b2_ref[...]
    out_ref[...] = _ln_elu(o, lw_ref[...], lb_ref[...])


def _tc3(p0, p1, d0, d1, sel, b2r, lw, lb):
    blk = lambda shp: pl.BlockSpec(shp, lambda i: (i, 0))
    whole = lambda shp: pl.BlockSpec(shp, lambda i: (0, 0))
    return pl.pallas_call(
        _tc3_body,
        grid=(GR,),
        in_specs=[blk((BR, HID)), blk((BR, HID)), blk((BR, 16)), blk((BR, 16)),
                  whole((16, HID)), whole((1, HID)), whole((1, HID)),
                  whole((1, HID))],
        out_specs=blk((BR, HID)),
        out_shape=jax.ShapeDtypeStruct((NP, HID), jnp.float32),
    )(p0, p1, d0, d1, sel, b2r, lw, lb)


CHA = 96
SGA = 4


def _att1_body(src3_h, dst3_h, as_h, ad_h, g_h, z_h,
               ex_h, den_h,
               den_sh, srcv2, dstv2, g1A, g2A, exA, g1B, g2B, exB,
               sA1, sA2, sB1, sB2, gv, wbv):
    c = lax.axis_index("c")
    s = lax.axis_index("s")
    w = s * NC + c
    pltpu.sync_copy(z_h, den_sh.at[pl.ds(s * STRIPE, STRIPE)])
    pltpu.sync_copy(g_h, gv)
    plsc.subcore_barrier()
    gvec = gv[...]
    NCHT = (EP // 32) // CHA
    crow = w * NCHT
    ebase = w * (EP // 32)

    def fire2(t, g1p, g2p, s1, s2):
        pltpu.async_copy(as_h.at[srcv2.at[t]], g1p, s1)
        pltpu.async_copy(ad_h.at[dstv2.at[t]], g2p, s2)

    def wait2(t, g1p, g2p, s1, s2):
        pltpu.make_async_copy(as_h.at[srcv2.at[t]], g1p, s1).wait()
        pltpu.make_async_copy(ad_h.at[dstv2.at[t]], g2p, s2).wait()

    def work(t, g1p, g2p, exp_):
        @pl.loop(0, CHA, unroll=2)
        def _edge(j):
            a = g1p[j, :] + g2p[j, :]
            a = jnp.where(a >= 0.0, a, 0.2 * a)
            exp_[j, :] = jnp.exp(a - gvec)

        pltpu.sync_copy(exp_, ex_h.at[pl.ds(ebase + t * CHA, CHA)])
        pltpu.sync_copy(exp_, den_sh.at[dstv2.at[t]], add=True)

    @pl.loop(0, NCHT // SGA)
    def _u(u):
        pltpu.sync_copy(src3_h.at[pl.ds(crow + u * SGA, SGA)], srcv2)
        pltpu.sync_copy(dst3_h.at[pl.ds(crow + u * SGA, SGA)], dstv2)
        fire2(0, g1A, g2A, sA1, sA2)
        for pp in range(SGA // 2):
            tA, tB = 2 * pp, 2 * pp + 1
            fire2(tB, g1B, g2B, sB1, sB2)
            wait2(tA, g1A, g2A, sA1, sA2)
            work(tA, g1A, g2A, exA)
            if pp < SGA // 2 - 1:
                fire2(tA + 2, g1A, g2A, sA1, sA2)
            wait2(tB, g1B, g2B, sB1, sB2)
            work(tB, g1B, g2B, exB)

    plsc.subcore_barrier()
    pltpu.sync_copy(den_sh.at[pl.ds(s * STRIPE, STRIPE)], wbv)
    pltpu.sync_copy(wbv, den_h.at[pl.ds(c * NP + s * STRIPE, STRIPE)])


def _att1(src3, dst3, as16, ad16, g16, z16):
    f32 = jnp.float32
    return pl.kernel(
        _att1_body,
        out_type=[jax.ShapeDtypeStruct((EP, 16), f32),
                  jax.ShapeDtypeStruct((2 * NP, 16), f32)],
        mesh=_mesh(),
        compiler_params=pltpu.CompilerParams(use_tc_tiling_on_sc=False),
        scratch_types=[
            pltpu.VMEM_SHARED((NP, 16), f32),
            pltpu.VMEM((SGA, CHA), jnp.int32),
            pltpu.VMEM((SGA, CHA), jnp.int32),
            pltpu.VMEM((CHA, 16), f32),
            pltpu.VMEM((CHA, 16), f32),
            pltpu.VMEM((CHA, 16), f32),
            pltpu.VMEM((CHA, 16), f32),
            pltpu.VMEM((CHA, 16), f32),
            pltpu.VMEM((CHA, 16), f32),
            pltpu.SemaphoreType.DMA,
            pltpu.SemaphoreType.DMA,
            pltpu.SemaphoreType.DMA,
            pltpu.SemaphoreType.DMA,
            pltpu.VMEM((16,), f32),
            pltpu.VMEM((STRIPE, 16), f32),
        ],
    )(src3, dst3, as16, ad16, g16, z16)


CHM = 128
SG = 6


def _fpass128(src2_h, dst2_h, ex_h, h_h, o_h, out_sh, s, f, z_h,
              srcv2, dstv2, exv6, rowsA, gsA, ssA, rowsB, gsB, wbv):
    NCHT = (EP // 16) // CHM
    pltpu.sync_copy(z_h, out_sh.at[pl.ds(s * STRIPE, STRIPE)])
    plsc.subcore_barrier()
    ebase = s * (EP // 16)
    crow = s * NCHT

    def scale(t, rowsp):
        @pl.loop(0, CHM, unroll=2)
        def _edge(j):
            exrow = exv6[t * CHM + j, :]
            for k in range(8):
                sc = exrow[f * 2 + (k // 4)]
                rowsp[j, pl.ds(k * 16, 16)] = rowsp[j, pl.ds(k * 16, 16)] * sc

    @pl.loop(0, NCHT // SG)
    def _u(u):
        pltpu.sync_copy(src2_h.at[pl.ds(crow + u * SG, SG)], srcv2)
        pltpu.sync_copy(dst2_h.at[pl.ds(crow + u * SG, SG)], dstv2)
        pltpu.sync_copy(ex_h.at[pl.ds(ebase + u * SG * CHM, SG * CHM)], exv6)
        pltpu.async_copy(h_h.at[srcv2.at[0]], rowsA, gsA)
        for p in range(SG // 2):
            tA, tB = 2 * p, 2 * p + 1
            pltpu.async_copy(h_h.at[srcv2.at[tB]], rowsB, gsB)
            pltpu.make_async_copy(h_h.at[srcv2.at[tA]], rowsA, gsA).wait()
            scale(tA, rowsA)
            pltpu.async_copy(rowsA, out_sh.at[dstv2.at[tA]], ssA, add=True)
            if p < SG // 2 - 1:
                pltpu.make_async_copy(rowsA, out_sh.at[dstv2.at[tA]],
                                      ssA).wait()
                pltpu.async_copy(h_h.at[srcv2.at[tA + 2]], rowsA, gsA)
            pltpu.make_async_copy(h_h.at[srcv2.at[tB]], rowsB, gsB).wait()
            scale(tB, rowsB)
            pltpu.sync_copy(rowsB, out_sh.at[dstv2.at[tB]], add=True)
        pltpu.make_async_copy(rowsA, out_sh.at[dstv2.at[SG - 2]], ssA).wait()

    plsc.subcore_barrier()
    rb = s * STRIPE
    for q in range(STRIPE // 16):
        pltpu.sync_copy(out_sh.at[pl.ds(rb + q * 16, 16)], wbv)
        pltpu.sync_copy(wbv, o_h.at[pl.ds(rb + q * 16, 16)])
    plsc.subcore_barrier()


def _msg1_body(src2_h, dst2_h, ex_h, hc0_h, hc1_h, hc2_h, hc3_h, z_h,
               oc0_h, oc1_h, oc2_h, oc3_h,
               out_sh, srcv2, dstv2, exv6, rowsA, rowsB,
               gsA, ssA, gsB, wbv):
    c = lax.axis_index("c")
    s = lax.axis_index("s")
    hs = (hc0_h, hc1_h, hc2_h, hc3_h)
    os_ = (oc0_h, oc1_h, oc2_h, oc3_h)
    for cc in range(NC):
        @pl.when(c == cc)
        def _(cc=cc):
            for fp in range(2):
                f = cc * 2 + fp
                _fpass128(src2_h, dst2_h, ex_h, hs[f], os_[f], out_sh, s, f,
                          z_h, srcv2, dstv2, exv6, rowsA, gsA, ssA,
                          rowsB, gsB, wbv)


def _msg1(src2, dst2, ex1, hc0, hc1, hc2, hc3, z128):
    f32 = jnp.float32
    return pl.kernel(
        _msg1_body,
        out_type=[jax.ShapeDtypeStruct((NP, 128), f32)] * 4,
        mesh=_mesh(),
        compiler_params=pltpu.CompilerParams(use_tc_tiling_on_sc=False),
        scratch_types=[
            pltpu.VMEM_SHARED((NP, 128), f32),
            pltpu.VMEM((SG, CHM), jnp.int32),
            pltpu.VMEM((SG, CHM), jnp.int32),
            pltpu.VMEM((SG * CHM, 16), f32),
            pltpu.VMEM((CHM, 128), f32),
            pltpu.VMEM((CHM, 128), f32),
            pltpu.SemaphoreType.DMA,
            pltpu.SemaphoreType.DMA,
            pltpu.SemaphoreType.DMA,
            pltpu.VMEM((16, 128), f32),
        ],
    )(src2, dst2, ex1, hc0, hc1, hc2, hc3, z128)


CHL = CHA
SGL = 4


def _l2_body(src3_h, dst3_h, as_h, ad_h, g_h, h2_h, z16_h, z64_h,
             op_h, dp_h,
             out_sh, den_sh, srcv2, dstv2,
             g1A, g2A, exA, rowsA, g1B, g2B, exB, rowsB,
             sA1, sA2, sAr, sB1, sB2, sBr, gv, wbv, wb64):
    c = lax.axis_index("c")
    s = lax.axis_index("s")
    w = s * NC + c
    pltpu.sync_copy(z16_h, den_sh.at[pl.ds(s * STRIPE, STRIPE)])
    pltpu.sync_copy(z64_h, out_sh.at[pl.ds(s * STRIPE, STRIPE)])
    pltpu.sync_copy(g_h, gv)
    plsc.subcore_barrier()
    gvec = gv[...]
    NCHT = (EP // 32) // CHL
    crow = w * NCHT

    def fire3(t, g1p, g2p, rowsp, s1, s2, sr):
        pltpu.async_copy(as_h.at[srcv2.at[t]], g1p, s1)
        pltpu.async_copy(ad_h.at[dstv2.at[t]], g2p, s2)
        pltpu.async_copy(h2_h.at[srcv2.at[t]], rowsp, sr)

    def wait3(t, g1p, g2p, rowsp, s1, s2, sr):
        pltpu.make_async_copy(as_h.at[srcv2.at[t]], g1p, s1).wait()
        pltpu.make_async_copy(ad_h.at[dstv2.at[t]], g2p, s2).wait()
        pltpu.make_async_copy(h2_h.at[srcv2.at[t]], rowsp, sr).wait()

    def work(t, g1p, g2p, exp_, rowsp):
        @pl.loop(0, CHL, unroll=2)
        def _edge(j):
            a = g1p[j, :] + g2p[j, :]
            a = jnp.where(a >= 0.0, a, 0.2 * a)
            e = jnp.exp(a - gvec)
            exp_[j, :] = e
            sc = e[0]
            for k in range(HID // 16):
                rowsp[j, pl.ds(k * 16, 16)] = rowsp[j, pl.ds(k * 16, 16)] * sc

        pltpu.sync_copy(exp_, den_sh.at[dstv2.at[t]], add=True)
        pltpu.sync_copy(rowsp, out_sh.at[dstv2.at[t]], add=True)

    @pl.loop(0, NCHT // SGL)
    def _u(u):
        pltpu.sync_copy(src3_h.at[pl.ds(crow + u * SGL, SGL)], srcv2)
        pltpu.sync_copy(dst3_h.at[pl.ds(crow + u * SGL, SGL)], dstv2)
        fire3(0, g1A, g2A, rowsA, sA1, sA2, sAr)
        for pp in range(SGL // 2):
            tA, tB = 2 * pp, 2 * pp + 1
            fire3(tB, g1B, g2B, rowsB, sB1, sB2, sBr)
            wait3(tA, g1A, g2A, rowsA, sA1, sA2, sAr)
            work(tA, g1A, g2A, exA, rowsA)
            if pp < SGL // 2 - 1:
                fire3(tA + 2, g1A, g2A, rowsA, sA1, sA2, sAr)
            wait3(tB, g1B, g2B, rowsB, sB1, sB2, sBr)
            work(tB, g1B, g2B, exB, rowsB)

    plsc.subcore_barrier()
    pltpu.sync_copy(den_sh.at[pl.ds(s * STRIPE, STRIPE)], wbv)
    pltpu.sync_copy(wbv, dp_h.at[pl.ds(c * NP + s * STRIPE, STRIPE)])
    rb = s * STRIPE
    for q in range(STRIPE // 16):
        pltpu.sync_copy(out_sh.at[pl.ds(rb + q * 16, 16)], wb64)
        pltpu.sync_copy(wb64, op_h.at[pl.ds(c * NP + rb + q * 16, 16)])


def _l2(src3, dst3, as2, ad2, g16, h2, z16, z64):
    f32 = jnp.float32
    return pl.kernel(
        _l2_body,
        out_type=[jax.ShapeDtypeStruct((2 * NP, HID), f32),
                  jax.ShapeDtypeStruct((2 * NP, 16), f32)],
        mesh=_mesh(),
        compiler_params=pltpu.CompilerParams(use_tc_tiling_on_sc=False),
        scratch_types=[
            pltpu.VMEM_SHARED((NP, HID), f32),
            pltpu.VMEM_SHARED((NP, 16), f32),
            pltpu.VMEM((SGL, CHL), jnp.int32),
            pltpu.VMEM((SGL, CHL), jnp.int32),
            pltpu.VMEM((CHL, 16), f32),
            pltpu.VMEM((CHL, 16), f32),
            pltpu.VMEM((CHL, 16), f32),
            pltpu.VMEM((CHL, HID), f32),
            pltpu.VMEM((CHL, 16), f32),
            pltpu.VMEM((CHL, 16), f32),
            pltpu.VMEM((CHL, 16), f32),
            pltpu.VMEM((CHL, HID), f32),
            pltpu.SemaphoreType.DMA,
            pltpu.SemaphoreType.DMA,
            pltpu.SemaphoreType.DMA,
            pltpu.SemaphoreType.DMA,
            pltpu.SemaphoreType.DMA,
            pltpu.SemaphoreType.DMA,
            pltpu.VMEM((16,), f32),
            pltpu.VMEM((STRIPE, 16), f32),
            pltpu.VMEM((16, HID), f32),
        ],
    )(src3, dst3, as2, ad2, g16, h2, z16, z64)


def _blockdiag16(att):
    h, d = att.shape
    eye = jnp.eye(8, dtype=att.dtype)[:h]
    m = att[:, :, None] * eye[:, None, :]
    m = m.reshape(h * d, 8)
    if h * d < 512:
        m = jnp.pad(m, ((0, 512 - h * d), (0, 0)))
    return jnp.concatenate([m, m], axis=1)


def kernel(x, edge_index, W1, att_src1, att_dst1, b1, ln1_w, ln1_b,
           W2, att_src2, att_dst2, b2, ln2_w, ln2_b):
    f32 = jnp.float32
    src = edge_index[0].astype(jnp.int32)
    dst = edge_index[1].astype(jnp.int32)
    loop = jnp.arange(N, dtype=jnp.int32)
    pad = jnp.full((EP - ET,), SENT, jnp.int32)
    src = jnp.concatenate([src, loop, pad])
    dst = jnp.concatenate([dst, loop, pad])

    xp = jnp.pad(x, ((0, NP - N), (0, 0)))

    ams1 = _blockdiag16(att_src1.reshape(HEADS, HID))
    amd1 = _blockdiag16(att_dst1.reshape(HEADS, HID))
    hc0, hc1, hc2, hc3, as1, ad1, gs1, gd1 = _tc1(xp, W1, ams1, amd1)
    g1 = jnp.max(gs1, axis=0) + jnp.max(gd1, axis=0)

    z16 = jnp.zeros((STRIPE, 16), f32)
    z64 = jnp.zeros((STRIPE, HID), f32)
    z128 = jnp.zeros((STRIPE, 128), f32)

    src2 = src.reshape(EP // CHM, CHM)
    dst2 = dst.reshape(EP // CHM, CHM)
    src3 = src.reshape(EP // CHA, CHA)
    dst3 = dst.reshape(EP // CHA, CHA)
    ex1, den1 = _att1(src3, dst3, as1, ad1, g1, z16)
    oc0, oc1, oc2, oc3 = _msg1(src2, dst2, ex1, hc0, hc1, hc2, hc3, z128)

    ex16 = jnp.concatenate(
        [jnp.repeat(jnp.eye(8, dtype=f32), HID, axis=1),
         jnp.zeros((8, 512), f32)], axis=0)
    a2s = jnp.tile(att_src2.reshape(HID, 1), (1, 16))
    a2d = jnp.tile(att_dst2.reshape(HID, 1), (1, 16))
    h2, as2, ad2, gs2, gd2 = _tc2(
        oc0, oc1, oc2, oc3, den1[:NP], den1[NP:], b1.reshape(1, 512),
        ln1_w.reshape(1, 512), ln1_b.reshape(1, 512), ex16, W2, a2s, a2d)
    g2 = jnp.max(gs2, axis=0) + jnp.max(gd2, axis=0)

    op2, dp2 = _l2(src3, dst3, as2, ad2, g2, h2, z16, z64)

    sel = jnp.zeros((16, HID), f32).at[0, :].set(1.0)
    out = _tc3(op2[:NP], op2[NP:], dp2[:NP], dp2[NP:], sel,
               b2.reshape(1, HID), ln2_w.reshape(1, HID),
               ln2_b.reshape(1, HID))
    return out[:N]
